# baseline (speedup 1.0000x reference)
import functools

import jax
import jax.numpy as jnp
from jax import lax
from jax.experimental import pallas as pl
from jax.experimental.pallas import tpu as pltpu
from jax.experimental.pallas import tpu_sc as plsc

N_NODES = 10000
N_EDGES = 320000
V_IN = 128
D = 128
EF = 16
H = 64
NC = 2
NS = 16
EB = 40
EDGES_PER_TILE = N_EDGES // NS
NBATCH = EDGES_PER_TILE // EB
NPAD = 10240
NODES_PER_TILE = NPAD // NS
EPI_CHUNK = 64
NODE_BLK = 400
EDGE_BLK = 3200


def _node_tables_body(x_ref, ws_ref, wd_ref, we_ref, src_ref, dst_ref, xe_ref):
    xb = x_ref[...]
    src_ref[0] = jnp.dot(xb, ws_ref[0], preferred_element_type=jnp.float32)
    src_ref[1] = jnp.dot(xb, ws_ref[1], preferred_element_type=jnp.float32)
    dst_ref[0] = jnp.dot(xb, wd_ref[0], preferred_element_type=jnp.float32)
    dst_ref[1] = jnp.dot(xb, wd_ref[1], preferred_element_type=jnp.float32)
    xe_ref[...] = jnp.dot(xb, we_ref[...], preferred_element_type=jnp.float32)


def _node_tables(x, ws, wd, we):
    nblk = N_NODES // NODE_BLK
    return pl.pallas_call(
        _node_tables_body,
        grid=(nblk,),
        in_specs=[
            pl.BlockSpec((NODE_BLK, V_IN), lambda i: (i, 0)),
            pl.BlockSpec((NC, V_IN, D), lambda i: (0, 0, 0)),
            pl.BlockSpec((NC, V_IN, D), lambda i: (0, 0, 0)),
            pl.BlockSpec((V_IN, D), lambda i: (0, 0)),
        ],
        out_specs=[
            pl.BlockSpec((NC, NODE_BLK, D), lambda i: (0, i, 0)),
            pl.BlockSpec((NC, NODE_BLK, D), lambda i: (0, i, 0)),
            pl.BlockSpec((NODE_BLK, D), lambda i: (i, 0)),
        ],
        out_shape=[
            jax.ShapeDtypeStruct((NC, N_NODES, D), jnp.float32),
            jax.ShapeDtypeStruct((NC, N_NODES, D), jnp.float32),
            jax.ShapeDtypeStruct((N_NODES, D), jnp.float32),
        ],
    )(x, ws, wd, we)


def _edge_tables_body(e_ref, wa_ref, wee_ref, edg_ref, ee_ref, mx_ref, mn_ref):
    eb = e_ref[...]
    o0 = jnp.dot(eb, wa_ref[0], preferred_element_type=jnp.float32)
    o1 = jnp.dot(eb, wa_ref[1], preferred_element_type=jnp.float32)
    edg_ref[0] = o0
    edg_ref[1] = o1
    ee_ref[...] = jnp.dot(eb, wee_ref[...], preferred_element_type=jnp.float32)
    acat = jnp.concatenate([o0[:, :H], o1[:, :H]], axis=1)
    mx_ref[0] = jnp.broadcast_to(jnp.max(acat, axis=0, keepdims=True), (8, D))
    mn_ref[0] = jnp.broadcast_to(jnp.min(acat, axis=0, keepdims=True), (8, D))


def _edge_tables(e, wa, wee):
    nblk = N_EDGES // EDGE_BLK
    return pl.pallas_call(
        _edge_tables_body,
        grid=(nblk,),
        in_specs=[
            pl.BlockSpec((EDGE_BLK, EF), lambda i: (i, 0)),
            pl.BlockSpec((NC, EF, D), lambda i: (0, 0, 0)),
            pl.BlockSpec((EF, EF), lambda i: (0, 0)),
        ],
        out_specs=[
            pl.BlockSpec((NC, EDGE_BLK, D), lambda i: (0, i, 0)),
            pl.BlockSpec((EDGE_BLK, EF), lambda i: (i, 0)),
            pl.BlockSpec((1, 8, D), lambda i: (i, 0, 0)),
            pl.BlockSpec((1, 8, D), lambda i: (i, 0, 0)),
        ],
        out_shape=[
            jax.ShapeDtypeStruct((NC, N_EDGES, D), jnp.float32),
            jax.ShapeDtypeStruct((N_EDGES, EF), jnp.float32),
            jax.ShapeDtypeStruct((nblk, 8, D), jnp.float32),
            jax.ShapeDtypeStruct((nblk, 8, D), jnp.float32),
        ],
    )(e, wa, wee)


def _sc_body(src_tab, dst_tab, edg_tab, xe_tab, ee_tab,
             big_idx,
             m_hbm, bt_hbm, pw_hbm,
             out_x, out_e,
             s_acc,
             bidxA, bidxB, dstsA, dstsB,
             srcrowsA, dstrowsA, edgrowsA,
             srcrowsB, dstrowsB, edgrowsB,
             scat, eerows, ebo,
             mvec, btvec, pwvec,
             semA, semB):
    ci = lax.axis_index("c")
    si = lax.axis_index("s")
    mbase = ci * H

    pltpu.sync_copy(m_hbm, mvec)
    pltpu.sync_copy(bt_hbm, btvec)
    pltpu.sync_copy(pw_hbm, pwvec)
    pwv = pwvec[...]
    zero16 = jnp.zeros((16,), jnp.float32)

    @pl.loop(0, EB * 8)
    def _zbody(i):
        r = lax.shift_right_logical(i, 3)
        co = jnp.bitwise_and(i, 7) * 16
        scat[r, pl.ds(co, 16)] = zero16

    for k in range(NODES_PER_TILE // EB):
        base = si * NODES_PER_TILE + k * EB
        pltpu.sync_copy(scat, s_acc.at[pl.ds(base, EB)])
    plsc.subcore_barrier()

    ebase = si * EDGES_PER_TILE
    idx_off = ci * N_EDGES
    mvs = [mvec[pl.ds(mbase + h * 16, 16)] for h in range(4)]

    def _copy40(dst_ref, src_ref, off):
        for c in (0, 16, 24):
            dst_ref[pl.ds(c, 16)] = src_ref[pl.ds(off + c, 16)]

    def _softmax_batch(rows_s, rows_d, rows_e, dsts):
        @pl.loop(0, EB, unroll=2)
        def _cbody(b):
            for h in range(4):
                co = h * 16
                a1 = rows_s[b, pl.ds(co, 16)]
                a2 = rows_d[b, pl.ds(co, 16)]
                ae = rows_e[b, pl.ds(co, 16)]
                lin = a1 + a2 + ae
                logit = jnp.where(lin >= 0.0, lin, pwv * lin)
                ex = jnp.exp(logit - mvs[h])
                t1 = rows_s[b, pl.ds(co + H, 16)]
                t2 = rows_d[b, pl.ds(co + H, 16)]
                te = rows_e[b, pl.ds(co + H, 16)]
                scat[b, pl.ds(co, 16)] = ex
                scat[b, pl.ds(co + H, 16)] = ex * (t1 + t2 + te)

        pltpu.sync_copy(scat, s_acc.at[dsts], add=True)

    @pl.loop(0, NBATCH // 2)
    def _pair(g):
        start = ebase + g * (2 * EB)
        prow = (ci * (N_EDGES // (2 * EB)) +
                si * (NBATCH // 2) + g) * (6 * EB)
        pltpu.sync_copy(big_idx.at[pl.ds(prow, 6 * EB)], bidxA)
        _copy40(dstsA, bidxA, 4 * EB)
        _copy40(dstsB, bidxA, 5 * EB)

        cpA1 = pltpu.async_copy(src_tab.at[bidxA.at[pl.ds(0, EB)]],
                                srcrowsA, semA)
        cpA2 = pltpu.async_copy(dst_tab.at[bidxA.at[pl.ds(2 * EB, EB)]],
                                dstrowsA, semA)
        cpA3 = pltpu.async_copy(edg_tab.at[pl.ds(idx_off + start, EB)],
                                edgrowsA, semA)
        cpB1 = pltpu.async_copy(src_tab.at[bidxA.at[pl.ds(EB, EB)]],
                                srcrowsB, semB)
        cpB2 = pltpu.async_copy(dst_tab.at[bidxA.at[pl.ds(3 * EB, EB)]],
                                dstrowsB, semB)
        cpB3 = pltpu.async_copy(edg_tab.at[pl.ds(idx_off + start + EB, EB)],
                                edgrowsB, semB)
        cpA1.wait()
        cpA2.wait()
        cpA3.wait()
        _softmax_batch(srcrowsA, dstrowsA, edgrowsA, dstsA)
        cpB1.wait()
        cpB2.wait()
        cpB3.wait()
        _softmax_batch(srcrowsB, dstrowsB, edgrowsB, dstsB)

    wid = si * NC + ci
    nbase = wid * (N_EDGES // (NC * NS))

    def _ne_batch(rows_s, rows_d, start):
        pltpu.sync_copy(ee_tab.at[pl.ds(start, EB)], eerows)

        @pl.loop(0, EB)
        def _nbody(b):
            eerows[b, :] = (rows_s[b, pl.ds(0, EF)] +
                            rows_d[b, pl.ds(0, EF)] + eerows[b, :])

        pltpu.sync_copy(eerows, out_e.at[pl.ds(start, EB)])

    @pl.loop(0, N_EDGES // (NC * NS * EB * 2))
    def _nepair(g):
        start = nbase + g * (2 * EB)
        p0 = (nbase // (2 * EB) + g) * (6 * EB)
        pltpu.sync_copy(big_idx.at[pl.ds(p0, 6 * EB)], bidxA)
        cpA1 = pltpu.async_copy(xe_tab.at[bidxA.at[pl.ds(0, EB)]],
                                srcrowsA, semA)
        cpA2 = pltpu.async_copy(xe_tab.at[bidxA.at[pl.ds(4 * EB, EB)]],
                                dstrowsA, semA)
        cpB1 = pltpu.async_copy(xe_tab.at[bidxA.at[pl.ds(EB, EB)]],
                                srcrowsB, semB)
        cpB2 = pltpu.async_copy(xe_tab.at[bidxA.at[pl.ds(5 * EB, EB)]],
                                dstrowsB, semB)
        cpA1.wait()
        cpA2.wait()
        _ne_batch(srcrowsA, dstrowsA, start)
        cpB1.wait()
        cpB2.wait()
        _ne_batch(srcrowsB, dstrowsB, start + EB)

    plsc.subcore_barrier()

    eps = jnp.full((16,), 1e-16, jnp.float32)
    bts = [btvec[pl.ds(mbase + h * 16, 16)] for h in range(4)]
    for k in range(NODES_PER_TILE // EB):
        base = si * NODES_PER_TILE + k * EB
        pltpu.sync_copy(s_acc.at[pl.ds(base, EB)], srcrowsA)

        @pl.loop(0, EB)
        def _ebody(r):
            for h in range(4):
                co = h * 16
                s0 = srcrowsA[r, pl.ds(co, 16)]
                s1 = srcrowsA[r, pl.ds(co + H, 16)]
                ebo[r, pl.ds(co, 16)] = s1 / (s0 + eps) + bts[h]

        pltpu.sync_copy(ebo, out_x.at[pl.ds(ci * NPAD + base, EB)])


_sc_pass = functools.partial(
    pl.kernel,
    out_type=[
        jax.ShapeDtypeStruct((NC * NPAD, H), jnp.float32),
        jax.ShapeDtypeStruct((N_EDGES, EF), jnp.float32),
    ],
    mesh=plsc.VectorSubcoreMesh(
        core_axis_name="c", subcore_axis_name="s", num_cores=NC,
        num_subcores=NS),
    scratch_types=[
        pltpu.VMEM_SHARED((NPAD, D), jnp.float32),
        pltpu.VMEM((6 * EB,), jnp.int32),
        pltpu.VMEM((6 * EB,), jnp.int32),
        pltpu.VMEM((EB,), jnp.int32),
        pltpu.VMEM((EB,), jnp.int32),
        pltpu.VMEM((EB, D), jnp.float32),
        pltpu.VMEM((EB, D), jnp.float32),
        pltpu.VMEM((EB, D), jnp.float32),
        pltpu.VMEM((EB, D), jnp.float32),
        pltpu.VMEM((EB, D), jnp.float32),
        pltpu.VMEM((EB, D), jnp.float32),
        pltpu.VMEM((EB, D), jnp.float32),
        pltpu.VMEM((EB, EF), jnp.float32),
        pltpu.VMEM((EB, H), jnp.float32),
        pltpu.VMEM((D,), jnp.float32),
        pltpu.VMEM((D,), jnp.float32),
        pltpu.VMEM((16,), jnp.float32),
        pltpu.SemaphoreType.DMA,
        pltpu.SemaphoreType.DMA,
    ],
)(_sc_body)


def kernel(x, edge_index, edge_attr, W_a, W_T, b_T, W_e, W_ee, prelu_w):
    x = x.astype(jnp.float32)
    e = edge_attr.astype(jnp.float32)
    src = edge_index[0].astype(jnp.int32)
    dst = edge_index[1].astype(jnp.int32)

    A2, Ae, A1 = W_a[:V_IN], W_a[V_IN:V_IN + EF], W_a[V_IN + EF:]
    T2, Te, T1 = W_T[:V_IN], W_T[V_IN:V_IN + EF], W_T[V_IN + EF:]

    def halves(a_part, t_part):
        return jnp.stack([
            jnp.concatenate([a_part[:, :H], t_part[:, :H]], axis=1),
            jnp.concatenate([a_part[:, H:], t_part[:, H:]], axis=1),
        ])

    ws = halves(A1, T1)
    wd = halves(A2, T2)
    wa = halves(Ae, Te)

    wep = jnp.zeros((V_IN, D), jnp.float32).at[:, :EF].set(W_e)
    src_pair, dst_pair, xe = _node_tables(x, ws, wd, wep)
    edg_pair, ee, amx, amn = _edge_tables(e, wa, W_ee)

    smax = jnp.concatenate([src_pair[0, :, :H].max(0), src_pair[1, :, :H].max(0)])
    smin = jnp.concatenate([src_pair[0, :, :H].min(0), src_pair[1, :, :H].min(0)])
    dmax = jnp.concatenate([dst_pair[0, :, :H].max(0), dst_pair[1, :, :H].max(0)])
    dmin = jnp.concatenate([dst_pair[0, :, :H].min(0), dst_pair[1, :, :H].min(0)])
    emax = amx.max(axis=(0, 1))
    emin = amn.min(axis=(0, 1))
    hi = smax + dmax + emax
    lo = smin + dmin + emin
    mvec = jnp.maximum(hi, jnp.maximum(prelu_w * hi, prelu_w * lo))
    mvec = mvec.astype(jnp.float32)

    src_tab = src_pair.reshape(NC * N_NODES, D)
    dst_tab = dst_pair.reshape(NC * N_NODES, D)
    edg_tab = edg_pair.reshape(NC * N_EDGES, D)
    pwv = jnp.full((16,), prelu_w, jnp.float32)
    srcp = src.reshape(-1, 2 * EB)
    dstp = dst.reshape(-1, 2 * EB)
    big = jnp.concatenate(
        [jnp.concatenate([srcp + h2 * N_NODES, dstp + h2 * N_NODES, dstp],
                         axis=1) for h2 in range(NC)]).reshape(-1)

    out_x, out_e = _sc_pass(src_tab, dst_tab, edg_tab, xe, ee, big,
                            mvec, b_T.astype(jnp.float32), pwv)

    new_x = jnp.concatenate([out_x[:N_NODES], out_x[NPAD:NPAD + N_NODES]],
                            axis=1)
    return (new_x, out_e)


# speedup vs baseline: 1.5152x; 1.5152x over previous
import functools

import jax
import jax.numpy as jnp
from jax import lax
from jax.experimental import pallas as pl
from jax.experimental.pallas import tpu as pltpu
from jax.experimental.pallas import tpu_sc as plsc

N_NODES = 10000
N_EDGES = 320000
V_IN = 128
D = 128
EF = 16
H = 64
NC = 2
NS = 16
EB = 40
EDGES_PER_TILE = N_EDGES // NS
NBATCH = EDGES_PER_TILE // EB
NPAD = 10240
NODES_PER_TILE = NPAD // NS
EPI_CHUNK = 64
NODE_BLK = 400
EDGE_BLK = 3200


def _node_tables_body(x_ref, ws_ref, wd_ref, we_ref, src_ref, dst_ref, xe_ref):
    xb = x_ref[...]
    src_ref[0] = jnp.dot(xb, ws_ref[0], preferred_element_type=jnp.float32)
    src_ref[1] = jnp.dot(xb, ws_ref[1], preferred_element_type=jnp.float32)
    dst_ref[0] = jnp.dot(xb, wd_ref[0], preferred_element_type=jnp.float32)
    dst_ref[1] = jnp.dot(xb, wd_ref[1], preferred_element_type=jnp.float32)
    xe_ref[...] = jnp.dot(xb, we_ref[...], preferred_element_type=jnp.float32)


def _node_tables(x, ws, wd, we):
    nblk = N_NODES // NODE_BLK
    return pl.pallas_call(
        _node_tables_body,
        grid=(nblk,),
        in_specs=[
            pl.BlockSpec((NODE_BLK, V_IN), lambda i: (i, 0)),
            pl.BlockSpec((NC, V_IN, D), lambda i: (0, 0, 0)),
            pl.BlockSpec((NC, V_IN, D), lambda i: (0, 0, 0)),
            pl.BlockSpec((V_IN, D), lambda i: (0, 0)),
        ],
        out_specs=[
            pl.BlockSpec((NC, NODE_BLK, D), lambda i: (0, i, 0)),
            pl.BlockSpec((NC, NODE_BLK, D), lambda i: (0, i, 0)),
            pl.BlockSpec((NODE_BLK, D), lambda i: (i, 0)),
        ],
        out_shape=[
            jax.ShapeDtypeStruct((NC, N_NODES, D), jnp.float32),
            jax.ShapeDtypeStruct((NC, N_NODES, D), jnp.float32),
            jax.ShapeDtypeStruct((N_NODES, D), jnp.float32),
        ],
    )(x, ws, wd, we)


def _edge_tables_body(e_ref, wa_ref, wee_ref, edg_ref, ee_ref, mx_ref, mn_ref):
    eb = e_ref[...]
    o0 = jnp.dot(eb, wa_ref[0], preferred_element_type=jnp.float32)
    o1 = jnp.dot(eb, wa_ref[1], preferred_element_type=jnp.float32)
    edg_ref[0] = o0
    edg_ref[1] = o1
    ee_ref[...] = jnp.dot(eb, wee_ref[...], preferred_element_type=jnp.float32)
    acat = jnp.concatenate([o0[:, :H], o1[:, :H]], axis=1)
    mx_ref[0] = jnp.broadcast_to(jnp.max(acat, axis=0, keepdims=True), (8, D))
    mn_ref[0] = jnp.broadcast_to(jnp.min(acat, axis=0, keepdims=True), (8, D))


def _edge_tables(e, wa, wee):
    nblk = N_EDGES // EDGE_BLK
    return pl.pallas_call(
        _edge_tables_body,
        grid=(nblk,),
        in_specs=[
            pl.BlockSpec((EDGE_BLK, EF), lambda i: (i, 0)),
            pl.BlockSpec((NC, EF, D), lambda i: (0, 0, 0)),
            pl.BlockSpec((EF, EF), lambda i: (0, 0)),
        ],
        out_specs=[
            pl.BlockSpec((NC, EDGE_BLK, D), lambda i: (0, i, 0)),
            pl.BlockSpec((EDGE_BLK, EF), lambda i: (i, 0)),
            pl.BlockSpec((1, 8, D), lambda i: (i, 0, 0)),
            pl.BlockSpec((1, 8, D), lambda i: (i, 0, 0)),
        ],
        out_shape=[
            jax.ShapeDtypeStruct((NC, N_EDGES, D), jnp.float32),
            jax.ShapeDtypeStruct((N_EDGES, EF), jnp.float32),
            jax.ShapeDtypeStruct((nblk, 8, D), jnp.float32),
            jax.ShapeDtypeStruct((nblk, 8, D), jnp.float32),
        ],
    )(e, wa, wee)


def _sc_body(src_tab, dst_tab, edg_tab, xe_tab, ee_tab,
             big_idx,
             m_hbm, bt_hbm, pw_hbm,
             out_x, out_e,
             s_acc,
             bidxA, bidxB, dstsA, dstsB,
             srcrowsA, dstrowsA, edgrowsA,
             srcrowsB, dstrowsB, edgrowsB,
             scat, eerows, ebo,
             mvec, btvec, pwvec,
             semA, semB):
    ci = lax.axis_index("c")
    si = lax.axis_index("s")
    mbase = ci * H

    pltpu.sync_copy(m_hbm, mvec)
    pltpu.sync_copy(bt_hbm, btvec)
    pltpu.sync_copy(pw_hbm, pwvec)
    pwv = pwvec[...]
    zero16 = jnp.zeros((16,), jnp.float32)

    @pl.loop(0, EB * 8)
    def _zbody(i):
        r = lax.shift_right_logical(i, 3)
        co = jnp.bitwise_and(i, 7) * 16
        scat[r, pl.ds(co, 16)] = zero16

    for k in range(NODES_PER_TILE // EB):
        base = si * NODES_PER_TILE + k * EB
        pltpu.sync_copy(scat, s_acc.at[pl.ds(base, EB)])
    plsc.subcore_barrier()

    ebase = si * EDGES_PER_TILE
    idx_off = ci * N_EDGES
    mvs = [mvec[pl.ds(mbase + h * 16, 16)] for h in range(4)]

    def _copy40(dst_ref, src_ref, off):
        for c in (0, 16, 24):
            dst_ref[pl.ds(c, 16)] = src_ref[pl.ds(off + c, 16)]

    def _softmax_batch(rows_s, rows_d, rows_e, dsts):
        @pl.loop(0, EB)
        def _cbody(b):
            for h in range(4):
                co = h * 16
                a1 = rows_s[b, pl.ds(co, 16)]
                a2 = rows_d[b, pl.ds(co, 16)]
                ae = rows_e[b, pl.ds(co, 16)]
                lin = a1 + a2 + ae
                logit = jnp.where(lin >= 0.0, lin, pwv * lin)
                ex = jnp.exp(logit - mvs[h])
                t1 = rows_s[b, pl.ds(co + H, 16)]
                t2 = rows_d[b, pl.ds(co + H, 16)]
                te = rows_e[b, pl.ds(co + H, 16)]
                scat[b, pl.ds(co, 16)] = ex
                scat[b, pl.ds(co + H, 16)] = ex * (t1 + t2 + te)

        pltpu.sync_copy(scat, s_acc.at[dsts], add=True)

    @pl.loop(0, NBATCH // 2)
    def _pair(g):
        start = ebase + g * (2 * EB)
        prow = (ci * (N_EDGES // (2 * EB)) +
                si * (NBATCH // 2) + g) * (6 * EB)
        pltpu.sync_copy(big_idx.at[pl.ds(prow, 6 * EB)], bidxA)
        _copy40(dstsA, bidxA, 4 * EB)
        _copy40(dstsB, bidxA, 5 * EB)

        cpA1 = pltpu.async_copy(src_tab.at[bidxA.at[pl.ds(0, EB)]],
                                srcrowsA, semA)
        cpA2 = pltpu.async_copy(dst_tab.at[bidxA.at[pl.ds(2 * EB, EB)]],
                                dstrowsA, semA)
        cpA3 = pltpu.async_copy(edg_tab.at[pl.ds(idx_off + start, EB)],
                                edgrowsA, semA)
        cpB1 = pltpu.async_copy(src_tab.at[bidxA.at[pl.ds(EB, EB)]],
                                srcrowsB, semB)
        cpB2 = pltpu.async_copy(dst_tab.at[bidxA.at[pl.ds(3 * EB, EB)]],
                                dstrowsB, semB)
        cpB3 = pltpu.async_copy(edg_tab.at[pl.ds(idx_off + start + EB, EB)],
                                edgrowsB, semB)
        cpA1.wait()
        cpA2.wait()
        cpA3.wait()
        _softmax_batch(srcrowsA, dstrowsA, edgrowsA, dstsA)
        cpB1.wait()
        cpB2.wait()
        cpB3.wait()
        _softmax_batch(srcrowsB, dstrowsB, edgrowsB, dstsB)

    wid = si * NC + ci
    nbase = wid * (N_EDGES // (NC * NS))

    def _ne_batch(rows_s, rows_d, start):
        pltpu.sync_copy(ee_tab.at[pl.ds(start, EB)], eerows)

        @pl.loop(0, EB)
        def _nbody(b):
            eerows[b, :] = (rows_s[b, pl.ds(0, EF)] +
                            rows_d[b, pl.ds(0, EF)] + eerows[b, :])

        pltpu.sync_copy(eerows, out_e.at[pl.ds(start, EB)])

    @pl.loop(0, N_EDGES // (NC * NS * EB * 2))
    def _nepair(g):
        start = nbase + g * (2 * EB)
        p0 = (nbase // (2 * EB) + g) * (6 * EB)
        pltpu.sync_copy(big_idx.at[pl.ds(p0, 6 * EB)], bidxA)
        cpA1 = pltpu.async_copy(xe_tab.at[bidxA.at[pl.ds(0, EB)]],
                                srcrowsA, semA)
        cpA2 = pltpu.async_copy(xe_tab.at[bidxA.at[pl.ds(4 * EB, EB)]],
                                dstrowsA, semA)
        cpB1 = pltpu.async_copy(xe_tab.at[bidxA.at[pl.ds(EB, EB)]],
                                srcrowsB, semB)
        cpB2 = pltpu.async_copy(xe_tab.at[bidxA.at[pl.ds(5 * EB, EB)]],
                                dstrowsB, semB)
        cpA1.wait()
        cpA2.wait()
        _ne_batch(srcrowsA, dstrowsA, start)
        cpB1.wait()
        cpB2.wait()
        _ne_batch(srcrowsB, dstrowsB, start + EB)

    plsc.subcore_barrier()

    eps = jnp.full((16,), 1e-16, jnp.float32)
    bts = [btvec[pl.ds(mbase + h * 16, 16)] for h in range(4)]
    for k in range(NODES_PER_TILE // EB):
        base = si * NODES_PER_TILE + k * EB
        pltpu.sync_copy(s_acc.at[pl.ds(base, EB)], srcrowsA)

        @pl.loop(0, EB)
        def _ebody(r):
            for h in range(4):
                co = h * 16
                s0 = srcrowsA[r, pl.ds(co, 16)]
                s1 = srcrowsA[r, pl.ds(co + H, 16)]
                ebo[r, pl.ds(co, 16)] = s1 / (s0 + eps) + bts[h]

        pltpu.sync_copy(ebo, out_x.at[pl.ds(ci * NPAD + base, EB)])


_sc_pass = functools.partial(
    pl.kernel,
    out_type=[
        jax.ShapeDtypeStruct((NC * NPAD, H), jnp.float32),
        jax.ShapeDtypeStruct((N_EDGES, EF), jnp.float32),
    ],
    mesh=plsc.VectorSubcoreMesh(
        core_axis_name="c", subcore_axis_name="s", num_cores=NC,
        num_subcores=NS),
    scratch_types=[
        pltpu.VMEM_SHARED((NPAD, D), jnp.float32),
        pltpu.VMEM((6 * EB,), jnp.int32),
        pltpu.VMEM((6 * EB,), jnp.int32),
        pltpu.VMEM((EB,), jnp.int32),
        pltpu.VMEM((EB,), jnp.int32),
        pltpu.VMEM((EB, D), jnp.float32),
        pltpu.VMEM((EB, D), jnp.float32),
        pltpu.VMEM((EB, D), jnp.float32),
        pltpu.VMEM((EB, D), jnp.float32),
        pltpu.VMEM((EB, D), jnp.float32),
        pltpu.VMEM((EB, D), jnp.float32),
        pltpu.VMEM((EB, D), jnp.float32),
        pltpu.VMEM((EB, EF), jnp.float32),
        pltpu.VMEM((EB, H), jnp.float32),
        pltpu.VMEM((D,), jnp.float32),
        pltpu.VMEM((D,), jnp.float32),
        pltpu.VMEM((16,), jnp.float32),
        pltpu.SemaphoreType.DMA,
        pltpu.SemaphoreType.DMA,
    ],
)(_sc_body)


def kernel(x, edge_index, edge_attr, W_a, W_T, b_T, W_e, W_ee, prelu_w):
    x = x.astype(jnp.float32)
    e = edge_attr.astype(jnp.float32)
    src = edge_index[0].astype(jnp.int32)
    dst = edge_index[1].astype(jnp.int32)

    A2, Ae, A1 = W_a[:V_IN], W_a[V_IN:V_IN + EF], W_a[V_IN + EF:]
    T2, Te, T1 = W_T[:V_IN], W_T[V_IN:V_IN + EF], W_T[V_IN + EF:]

    def halves(a_part, t_part):
        return jnp.stack([
            jnp.concatenate([a_part[:, :H], t_part[:, :H]], axis=1),
            jnp.concatenate([a_part[:, H:], t_part[:, H:]], axis=1),
        ])

    ws = halves(A1, T1)
    wd = halves(A2, T2)
    wa = halves(Ae, Te)

    wep = jnp.zeros((V_IN, D), jnp.float32).at[:, :EF].set(W_e)
    src_pair, dst_pair, xe = _node_tables(x, ws, wd, wep)
    edg_pair, ee, amx, amn = _edge_tables(e, wa, W_ee)

    smax = jnp.concatenate([src_pair[0, :, :H].max(0), src_pair[1, :, :H].max(0)])
    smin = jnp.concatenate([src_pair[0, :, :H].min(0), src_pair[1, :, :H].min(0)])
    dmax = jnp.concatenate([dst_pair[0, :, :H].max(0), dst_pair[1, :, :H].max(0)])
    dmin = jnp.concatenate([dst_pair[0, :, :H].min(0), dst_pair[1, :, :H].min(0)])
    emax = amx.max(axis=(0, 1))
    emin = amn.min(axis=(0, 1))
    hi = smax + dmax + emax
    lo = smin + dmin + emin
    mvec = jnp.maximum(hi, jnp.maximum(prelu_w * hi, prelu_w * lo))
    mvec = mvec.astype(jnp.float32)

    src_tab = src_pair.reshape(NC * N_NODES, D)
    dst_tab = dst_pair.reshape(NC * N_NODES, D)
    edg_tab = edg_pair.reshape(NC * N_EDGES, D)
    pwv = jnp.full((16,), prelu_w, jnp.float32)
    srcp = src.reshape(-1, 2 * EB)
    dstp = dst.reshape(-1, 2 * EB)
    big = jnp.concatenate(
        [jnp.concatenate([srcp + h2 * N_NODES, dstp + h2 * N_NODES, dstp],
                         axis=1) for h2 in range(NC)]).reshape(-1)

    out_x, out_e = _sc_pass(src_tab, dst_tab, edg_tab, xe, ee, big,
                            mvec, b_T.astype(jnp.float32), pwv)

    new_x = jnp.concatenate([out_x[:N_NODES], out_x[NPAD:NPAD + N_NODES]],
                            axis=1)
    return (new_x, out_e)


# speedup vs baseline: 1.8876x; 1.2458x over previous
import functools

import jax
import jax.numpy as jnp
from jax import lax
from jax.experimental import pallas as pl
from jax.experimental.pallas import tpu as pltpu
from jax.experimental.pallas import tpu_sc as plsc

N_NODES = 10000
N_EDGES = 320000
V_IN = 128
D = 128
EF = 16
H = 64
NC = 2
NS = 16
EB = 40
EDGES_PER_TILE = N_EDGES // NS
NBATCH = EDGES_PER_TILE // EB
NPAD = 10240
NODES_PER_TILE = NPAD // NS
EPI_CHUNK = 64
NODE_BLK = 400
EDGE_BLK = 3200


def _node_tables_body(x_ref, ws_ref, wd_ref, we_ref, src_ref, dst_ref, xe_ref):
    xb = x_ref[...]
    src_ref[0] = jnp.dot(xb, ws_ref[0], preferred_element_type=jnp.float32)
    src_ref[1] = jnp.dot(xb, ws_ref[1], preferred_element_type=jnp.float32)
    dst_ref[0] = jnp.dot(xb, wd_ref[0], preferred_element_type=jnp.float32)
    dst_ref[1] = jnp.dot(xb, wd_ref[1], preferred_element_type=jnp.float32)
    xe_ref[...] = jnp.dot(xb, we_ref[...], preferred_element_type=jnp.float32)


def _node_tables(x, ws, wd, we):
    nblk = N_NODES // NODE_BLK
    return pl.pallas_call(
        _node_tables_body,
        grid=(nblk,),
        in_specs=[
            pl.BlockSpec((NODE_BLK, V_IN), lambda i: (i, 0)),
            pl.BlockSpec((NC, V_IN, D), lambda i: (0, 0, 0)),
            pl.BlockSpec((NC, V_IN, D), lambda i: (0, 0, 0)),
            pl.BlockSpec((V_IN, D), lambda i: (0, 0)),
        ],
        out_specs=[
            pl.BlockSpec((NC, NODE_BLK, D), lambda i: (0, i, 0)),
            pl.BlockSpec((NC, NODE_BLK, D), lambda i: (0, i, 0)),
            pl.BlockSpec((NODE_BLK, D), lambda i: (i, 0)),
        ],
        out_shape=[
            jax.ShapeDtypeStruct((NC, N_NODES, D), jnp.float32),
            jax.ShapeDtypeStruct((NC, N_NODES, D), jnp.float32),
            jax.ShapeDtypeStruct((N_NODES, D), jnp.float32),
        ],
    )(x, ws, wd, we)


def _edge_tables_body(e_ref, wa_ref, wee_ref, edg_ref, ee_ref, mx_ref, mn_ref):
    eb = e_ref[...]
    o0 = jnp.dot(eb, wa_ref[0], preferred_element_type=jnp.float32)
    o1 = jnp.dot(eb, wa_ref[1], preferred_element_type=jnp.float32)
    edg_ref[0] = o0
    edg_ref[1] = o1
    ee_ref[...] = jnp.dot(eb, wee_ref[...], preferred_element_type=jnp.float32)
    acat = jnp.concatenate([o0[:, :H], o1[:, :H]], axis=1)
    mx_ref[0] = jnp.broadcast_to(jnp.max(acat, axis=0, keepdims=True), (8, D))
    mn_ref[0] = jnp.broadcast_to(jnp.min(acat, axis=0, keepdims=True), (8, D))


def _edge_tables(e, wa, wee):
    nblk = N_EDGES // EDGE_BLK
    return pl.pallas_call(
        _edge_tables_body,
        grid=(nblk,),
        in_specs=[
            pl.BlockSpec((EDGE_BLK, EF), lambda i: (i, 0)),
            pl.BlockSpec((NC, EF, D), lambda i: (0, 0, 0)),
            pl.BlockSpec((EF, EF), lambda i: (0, 0)),
        ],
        out_specs=[
            pl.BlockSpec((NC, EDGE_BLK, D), lambda i: (0, i, 0)),
            pl.BlockSpec((EDGE_BLK, EF), lambda i: (i, 0)),
            pl.BlockSpec((1, 8, D), lambda i: (i, 0, 0)),
            pl.BlockSpec((1, 8, D), lambda i: (i, 0, 0)),
        ],
        out_shape=[
            jax.ShapeDtypeStruct((NC, N_EDGES, D), jnp.float32),
            jax.ShapeDtypeStruct((N_EDGES, EF), jnp.float32),
            jax.ShapeDtypeStruct((nblk, 8, D), jnp.float32),
            jax.ShapeDtypeStruct((nblk, 8, D), jnp.float32),
        ],
    )(e, wa, wee)


def _sc_body(src_tab, dst_tab, edg_tab, xe_tab, ee_tab,
             big_idx,
             m_hbm, bt_hbm, pw_hbm,
             out_x, out_e,
             s_acc,
             bidxA, bidxB, dstsA, dstsB,
             srcrowsA, dstrowsA, edgrowsA,
             srcrowsB, dstrowsB, edgrowsB,
             scat, eerows, ebo,
             mvec, btvec, pwvec,
             semA, semB):
    ci = lax.axis_index("c")
    si = lax.axis_index("s")
    mbase = ci * H

    pltpu.sync_copy(m_hbm, mvec)
    pltpu.sync_copy(bt_hbm, btvec)
    pltpu.sync_copy(pw_hbm, pwvec)
    pwv = pwvec[...]
    zero16 = jnp.zeros((16,), jnp.float32)

    @pl.loop(0, EB * 8)
    def _zbody(i):
        r = lax.shift_right_logical(i, 3)
        co = jnp.bitwise_and(i, 7) * 16
        scat[r, pl.ds(co, 16)] = zero16

    for k in range(NODES_PER_TILE // EB):
        base = si * NODES_PER_TILE + k * EB
        pltpu.sync_copy(scat, s_acc.at[pl.ds(base, EB)])
    plsc.subcore_barrier()

    ebase = si * EDGES_PER_TILE
    idx_off = ci * N_EDGES
    mvs = [mvec[pl.ds(mbase + h * 16, 16)] for h in range(4)]

    def _copy40(dst_ref, src_ref, off):
        for c in (0, 16, 24):
            dst_ref[pl.ds(c, 16)] = src_ref[pl.ds(off + c, 16)]

    def _softmax_batch(rows_s, rows_d, rows_e, dsts, sem):
        pltpu.make_async_copy(src_tab.at[bidxA.at[pl.ds(0, EB)]],
                              rows_s, sem).wait()
        pltpu.make_async_copy(src_tab.at[bidxA.at[pl.ds(0, EB)]],
                              rows_d, sem).wait()
        pltpu.make_async_copy(src_tab.at[bidxA.at[pl.ds(0, EB)]],
                              rows_e, sem).wait()

        @pl.loop(0, EB)
        def _cbody(b):
            for h in range(4):
                co = h * 16
                a1 = rows_s[b, pl.ds(co, 16)]
                a2 = rows_d[b, pl.ds(co, 16)]
                ae = rows_e[b, pl.ds(co, 16)]
                lin = a1 + a2 + ae
                logit = jnp.where(lin >= 0.0, lin, pwv * lin)
                ex = jnp.exp(logit - mvs[h])
                t1 = rows_s[b, pl.ds(co + H, 16)]
                t2 = rows_d[b, pl.ds(co + H, 16)]
                te = rows_e[b, pl.ds(co + H, 16)]
                scat[b, pl.ds(co, 16)] = ex
                scat[b, pl.ds(co + H, 16)] = ex * (t1 + t2 + te)

        pltpu.sync_copy(scat, s_acc.at[dsts], add=True)

    NPAIR_T = NBATCH // 2
    prow0 = (ci * (N_EDGES // (2 * EB)) + si * NPAIR_T) * (6 * EB)

    def _issue(bidx, pstart, batch, rows_s, rows_d, rows_e, sem):
        cp1 = pltpu.async_copy(src_tab.at[bidx.at[pl.ds(batch * EB, EB)]],
                               rows_s, sem)
        cp2 = pltpu.async_copy(
            dst_tab.at[bidx.at[pl.ds((2 + batch) * EB, EB)]], rows_d, sem)
        cp3 = pltpu.async_copy(
            edg_tab.at[pl.ds(idx_off + pstart + batch * EB, EB)], rows_e, sem)
        return cp1, cp2, cp3

    pltpu.sync_copy(big_idx.at[pl.ds(prow0, 6 * EB)], bidxA)
    _issue(bidxA, ebase, 0, srcrowsA, dstrowsA, edgrowsA, semA)

    @pl.loop(0, NPAIR_T // 2)
    def _pairpair(gg):
        start0 = ebase + gg * (4 * EB)
        for half in range(2):
            bidx = (bidxA, bidxB)[half]
            bidx_next = (bidxB, bidxA)[half]
            pstart = start0 + half * (2 * EB)
            _issue(bidx, pstart, 1, srcrowsB, dstrowsB, edgrowsB, semB)
            _copy40(dstsA, bidx, 4 * EB)
            _copy40(dstsB, bidx, 5 * EB)
            prow_n = prow0 + (gg * 2 + half + 1) * (6 * EB)
            pltpu.sync_copy(big_idx.at[pl.ds(prow_n, 6 * EB)], bidx_next)
            _softmax_batch(srcrowsA, dstrowsA, edgrowsA, dstsA, semA)
            _issue(bidx_next, pstart + 2 * EB, 0,
                   srcrowsA, dstrowsA, edgrowsA, semA)
            _softmax_batch(srcrowsB, dstrowsB, edgrowsB, dstsB, semB)

    pltpu.make_async_copy(src_tab.at[bidxA.at[pl.ds(0, EB)]],
                          srcrowsA, semA).wait()
    pltpu.make_async_copy(dst_tab.at[bidxA.at[pl.ds(2 * EB, EB)]],
                          dstrowsA, semA).wait()
    pltpu.make_async_copy(edg_tab.at[pl.ds(idx_off + ebase, EB)],
                          edgrowsA, semA).wait()

    wid = si * NC + ci
    nbase = wid * (N_EDGES // (NC * NS))

    def _ne_batch(rows_s, rows_d, start):
        pltpu.sync_copy(ee_tab.at[pl.ds(start, EB)], eerows)

        @pl.loop(0, EB)
        def _nbody(b):
            eerows[b, :] = (rows_s[b, pl.ds(0, EF)] +
                            rows_d[b, pl.ds(0, EF)] + eerows[b, :])

        pltpu.sync_copy(eerows, out_e.at[pl.ds(start, EB)])

    @pl.loop(0, N_EDGES // (NC * NS * EB * 2))
    def _nepair(g):
        start = nbase + g * (2 * EB)
        p0 = (nbase // (2 * EB) + g) * (6 * EB)
        pltpu.sync_copy(big_idx.at[pl.ds(p0, 6 * EB)], bidxA)
        cpA1 = pltpu.async_copy(xe_tab.at[bidxA.at[pl.ds(0, EB)]],
                                srcrowsA, semA)
        cpA2 = pltpu.async_copy(xe_tab.at[bidxA.at[pl.ds(4 * EB, EB)]],
                                dstrowsA, semA)
        cpB1 = pltpu.async_copy(xe_tab.at[bidxA.at[pl.ds(EB, EB)]],
                                srcrowsB, semB)
        cpB2 = pltpu.async_copy(xe_tab.at[bidxA.at[pl.ds(5 * EB, EB)]],
                                dstrowsB, semB)
        cpA1.wait()
        cpA2.wait()
        _ne_batch(srcrowsA, dstrowsA, start)
        cpB1.wait()
        cpB2.wait()
        _ne_batch(srcrowsB, dstrowsB, start + EB)

    plsc.subcore_barrier()

    eps = jnp.full((16,), 1e-16, jnp.float32)
    bts = [btvec[pl.ds(mbase + h * 16, 16)] for h in range(4)]
    for k in range(NODES_PER_TILE // EB):
        base = si * NODES_PER_TILE + k * EB
        pltpu.sync_copy(s_acc.at[pl.ds(base, EB)], srcrowsA)

        @pl.loop(0, EB)
        def _ebody(r):
            for h in range(4):
                co = h * 16
                s0 = srcrowsA[r, pl.ds(co, 16)]
                s1 = srcrowsA[r, pl.ds(co + H, 16)]
                ebo[r, pl.ds(co, 16)] = s1 / (s0 + eps) + bts[h]

        pltpu.sync_copy(ebo, out_x.at[pl.ds(ci * NPAD + base, EB)])


_sc_pass = functools.partial(
    pl.kernel,
    out_type=[
        jax.ShapeDtypeStruct((NC * NPAD, H), jnp.float32),
        jax.ShapeDtypeStruct((N_EDGES, EF), jnp.float32),
    ],
    mesh=plsc.VectorSubcoreMesh(
        core_axis_name="c", subcore_axis_name="s", num_cores=NC,
        num_subcores=NS),
    scratch_types=[
        pltpu.VMEM_SHARED((NPAD, D), jnp.float32),
        pltpu.VMEM((6 * EB,), jnp.int32),
        pltpu.VMEM((6 * EB,), jnp.int32),
        pltpu.VMEM((EB,), jnp.int32),
        pltpu.VMEM((EB,), jnp.int32),
        pltpu.VMEM((EB, D), jnp.float32),
        pltpu.VMEM((EB, D), jnp.float32),
        pltpu.VMEM((EB, D), jnp.float32),
        pltpu.VMEM((EB, D), jnp.float32),
        pltpu.VMEM((EB, D), jnp.float32),
        pltpu.VMEM((EB, D), jnp.float32),
        pltpu.VMEM((EB, D), jnp.float32),
        pltpu.VMEM((EB, EF), jnp.float32),
        pltpu.VMEM((EB, H), jnp.float32),
        pltpu.VMEM((D,), jnp.float32),
        pltpu.VMEM((D,), jnp.float32),
        pltpu.VMEM((16,), jnp.float32),
        pltpu.SemaphoreType.DMA,
        pltpu.SemaphoreType.DMA,
    ],
)(_sc_body)


def kernel(x, edge_index, edge_attr, W_a, W_T, b_T, W_e, W_ee, prelu_w):
    x = x.astype(jnp.float32)
    e = edge_attr.astype(jnp.float32)
    src = edge_index[0].astype(jnp.int32)
    dst = edge_index[1].astype(jnp.int32)

    A2, Ae, A1 = W_a[:V_IN], W_a[V_IN:V_IN + EF], W_a[V_IN + EF:]
    T2, Te, T1 = W_T[:V_IN], W_T[V_IN:V_IN + EF], W_T[V_IN + EF:]

    def halves(a_part, t_part):
        return jnp.stack([
            jnp.concatenate([a_part[:, :H], t_part[:, :H]], axis=1),
            jnp.concatenate([a_part[:, H:], t_part[:, H:]], axis=1),
        ])

    ws = halves(A1, T1)
    wd = halves(A2, T2)
    wa = halves(Ae, Te)

    wep = jnp.zeros((V_IN, D), jnp.float32).at[:, :EF].set(W_e)
    src_pair, dst_pair, xe = _node_tables(x, ws, wd, wep)
    edg_pair, ee, amx, amn = _edge_tables(e, wa, W_ee)

    smax = jnp.concatenate([src_pair[0, :, :H].max(0), src_pair[1, :, :H].max(0)])
    smin = jnp.concatenate([src_pair[0, :, :H].min(0), src_pair[1, :, :H].min(0)])
    dmax = jnp.concatenate([dst_pair[0, :, :H].max(0), dst_pair[1, :, :H].max(0)])
    dmin = jnp.concatenate([dst_pair[0, :, :H].min(0), dst_pair[1, :, :H].min(0)])
    emax = amx.max(axis=(0, 1))
    emin = amn.min(axis=(0, 1))
    hi = smax + dmax + emax
    lo = smin + dmin + emin
    mvec = jnp.maximum(hi, jnp.maximum(prelu_w * hi, prelu_w * lo))
    mvec = mvec.astype(jnp.float32)

    src_tab = src_pair.reshape(NC * N_NODES, D)
    dst_tab = dst_pair.reshape(NC * N_NODES, D)
    edg_tab = edg_pair.reshape(NC * N_EDGES, D)
    pwv = jnp.full((16,), prelu_w, jnp.float32)
    srcp = src.reshape(-1, 2 * EB)
    dstp = dst.reshape(-1, 2 * EB)
    big = jnp.concatenate(
        [jnp.concatenate([srcp + h2 * N_NODES, dstp + h2 * N_NODES, dstp],
                         axis=1) for h2 in range(NC)]).reshape(-1)
    big = jnp.concatenate([big, jnp.zeros((6 * EB,), jnp.int32)])

    out_x, out_e = _sc_pass(src_tab, dst_tab, edg_tab, xe, ee, big,
                            mvec, b_T.astype(jnp.float32), pwv)

    new_x = jnp.concatenate([out_x[:N_NODES], out_x[NPAD:NPAD + N_NODES]],
                            axis=1)
    return (new_x, out_e)


# speedup vs baseline: 1.9143x; 1.0141x over previous
import functools

import jax
import jax.numpy as jnp
from jax import lax
from jax.experimental import pallas as pl
from jax.experimental.pallas import tpu as pltpu
from jax.experimental.pallas import tpu_sc as plsc

N_NODES = 10000
N_EDGES = 320000
V_IN = 128
D = 128
EF = 16
H = 64
NC = 2
NS = 16
EB = 40
EDGES_PER_TILE = N_EDGES // NS
NBATCH = EDGES_PER_TILE // EB
NPAD = 10240
NODES_PER_TILE = NPAD // NS
EPI_CHUNK = 64
NODE_BLK = 2000
EDGE_BLK = 8000


def _node_tables_body(x_ref, ws_ref, wd_ref, we_ref, src_ref, dst_ref, xe_ref):
    xb = x_ref[...]
    src_ref[0] = jnp.dot(xb, ws_ref[0], preferred_element_type=jnp.float32)
    src_ref[1] = jnp.dot(xb, ws_ref[1], preferred_element_type=jnp.float32)
    dst_ref[0] = jnp.dot(xb, wd_ref[0], preferred_element_type=jnp.float32)
    dst_ref[1] = jnp.dot(xb, wd_ref[1], preferred_element_type=jnp.float32)
    xe_ref[...] = jnp.dot(xb, we_ref[...], preferred_element_type=jnp.float32)


def _node_tables(x, ws, wd, we):
    nblk = N_NODES // NODE_BLK
    return pl.pallas_call(
        _node_tables_body,
        grid=(nblk,),
        in_specs=[
            pl.BlockSpec((NODE_BLK, V_IN), lambda i: (i, 0)),
            pl.BlockSpec((NC, V_IN, D), lambda i: (0, 0, 0)),
            pl.BlockSpec((NC, V_IN, D), lambda i: (0, 0, 0)),
            pl.BlockSpec((V_IN, D), lambda i: (0, 0)),
        ],
        out_specs=[
            pl.BlockSpec((NC, NODE_BLK, D), lambda i: (0, i, 0)),
            pl.BlockSpec((NC, NODE_BLK, D), lambda i: (0, i, 0)),
            pl.BlockSpec((NODE_BLK, D), lambda i: (i, 0)),
        ],
        out_shape=[
            jax.ShapeDtypeStruct((NC, N_NODES, D), jnp.float32),
            jax.ShapeDtypeStruct((NC, N_NODES, D), jnp.float32),
            jax.ShapeDtypeStruct((N_NODES, D), jnp.float32),
        ],
    )(x, ws, wd, we)


def _edge_tables_body(e_ref, wa_ref, wee_ref, edg_ref, ee_ref, mx_ref, mn_ref):
    eb = e_ref[...]
    o0 = jnp.dot(eb, wa_ref[0], preferred_element_type=jnp.float32)
    o1 = jnp.dot(eb, wa_ref[1], preferred_element_type=jnp.float32)
    edg_ref[0] = o0
    edg_ref[1] = o1
    ee_ref[...] = jnp.dot(eb, wee_ref[...], preferred_element_type=jnp.float32)
    acat = jnp.concatenate([o0[:, :H], o1[:, :H]], axis=1)
    mx_ref[0] = jnp.broadcast_to(jnp.max(acat, axis=0, keepdims=True), (8, D))
    mn_ref[0] = jnp.broadcast_to(jnp.min(acat, axis=0, keepdims=True), (8, D))


def _edge_tables(e, wa, wee):
    nblk = N_EDGES // EDGE_BLK
    return pl.pallas_call(
        _edge_tables_body,
        grid=(nblk,),
        in_specs=[
            pl.BlockSpec((EDGE_BLK, EF), lambda i: (i, 0)),
            pl.BlockSpec((NC, EF, D), lambda i: (0, 0, 0)),
            pl.BlockSpec((EF, EF), lambda i: (0, 0)),
        ],
        out_specs=[
            pl.BlockSpec((NC, EDGE_BLK, D), lambda i: (0, i, 0)),
            pl.BlockSpec((EDGE_BLK, EF), lambda i: (i, 0)),
            pl.BlockSpec((1, 8, D), lambda i: (i, 0, 0)),
            pl.BlockSpec((1, 8, D), lambda i: (i, 0, 0)),
        ],
        out_shape=[
            jax.ShapeDtypeStruct((NC, N_EDGES, D), jnp.float32),
            jax.ShapeDtypeStruct((N_EDGES, EF), jnp.float32),
            jax.ShapeDtypeStruct((nblk, 8, D), jnp.float32),
            jax.ShapeDtypeStruct((nblk, 8, D), jnp.float32),
        ],
    )(e, wa, wee)


def _sc_body(src_tab, dst_tab, edg_tab, xe_tab, ee_tab,
             big_idx,
             m_hbm, bt_hbm, pw_hbm,
             out_x, out_e,
             s_acc,
             bidxA, bidxB, dstsA, dstsB,
             srcrowsA, dstrowsA, edgrowsA,
             srcrowsB, dstrowsB, edgrowsB,
             scat, eerows, ebo,
             mvec, btvec, pwvec,
             semA, semB):
    ci = lax.axis_index("c")
    si = lax.axis_index("s")
    mbase = ci * H

    pltpu.sync_copy(m_hbm, mvec)
    pltpu.sync_copy(bt_hbm, btvec)
    pltpu.sync_copy(pw_hbm, pwvec)
    pwv = pwvec[...]
    zero16 = jnp.zeros((16,), jnp.float32)

    @pl.loop(0, EB * 8)
    def _zbody(i):
        r = lax.shift_right_logical(i, 3)
        co = jnp.bitwise_and(i, 7) * 16
        scat[r, pl.ds(co, 16)] = zero16

    for k in range(NODES_PER_TILE // EB):
        base = si * NODES_PER_TILE + k * EB
        pltpu.sync_copy(scat, s_acc.at[pl.ds(base, EB)])
    plsc.subcore_barrier()

    ebase = si * EDGES_PER_TILE
    idx_off = ci * N_EDGES
    mvs = [mvec[pl.ds(mbase + h * 16, 16)] for h in range(4)]

    def _copy40(dst_ref, src_ref, off):
        for c in (0, 16, 24):
            dst_ref[pl.ds(c, 16)] = src_ref[pl.ds(off + c, 16)]

    def _softmax_batch(rows_s, rows_d, rows_e, dsts, sem):
        pltpu.make_async_copy(src_tab.at[bidxA.at[pl.ds(0, EB)]],
                              rows_s, sem).wait()
        pltpu.make_async_copy(src_tab.at[bidxA.at[pl.ds(0, EB)]],
                              rows_d, sem).wait()
        pltpu.make_async_copy(src_tab.at[bidxA.at[pl.ds(0, EB)]],
                              rows_e, sem).wait()

        @pl.loop(0, EB)
        def _cbody(b):
            for h in range(4):
                co = h * 16
                a1 = rows_s[b, pl.ds(co, 16)]
                a2 = rows_d[b, pl.ds(co, 16)]
                ae = rows_e[b, pl.ds(co, 16)]
                lin = a1 + a2 + ae
                logit = jnp.where(lin >= 0.0, lin, pwv * lin)
                ex = jnp.exp(logit - mvs[h])
                t1 = rows_s[b, pl.ds(co + H, 16)]
                t2 = rows_d[b, pl.ds(co + H, 16)]
                te = rows_e[b, pl.ds(co + H, 16)]
                scat[b, pl.ds(co, 16)] = ex
                scat[b, pl.ds(co + H, 16)] = ex * (t1 + t2 + te)

        pltpu.sync_copy(scat, s_acc.at[dsts], add=True)

    NPAIR_T = NBATCH // 2
    prow0 = (ci * (N_EDGES // (2 * EB)) + si * NPAIR_T) * (6 * EB)

    def _issue(bidx, pstart, batch, rows_s, rows_d, rows_e, sem):
        cp1 = pltpu.async_copy(src_tab.at[bidx.at[pl.ds(batch * EB, EB)]],
                               rows_s, sem)
        cp2 = pltpu.async_copy(
            dst_tab.at[bidx.at[pl.ds((2 + batch) * EB, EB)]], rows_d, sem)
        cp3 = pltpu.async_copy(
            edg_tab.at[pl.ds(idx_off + pstart + batch * EB, EB)], rows_e, sem)
        return cp1, cp2, cp3

    pltpu.sync_copy(big_idx.at[pl.ds(prow0, 6 * EB)], bidxA)
    _issue(bidxA, ebase, 0, srcrowsA, dstrowsA, edgrowsA, semA)

    @pl.loop(0, NPAIR_T // 2)
    def _pairpair(gg):
        start0 = ebase + gg * (4 * EB)
        for half in range(2):
            bidx = (bidxA, bidxB)[half]
            bidx_next = (bidxB, bidxA)[half]
            pstart = start0 + half * (2 * EB)
            _issue(bidx, pstart, 1, srcrowsB, dstrowsB, edgrowsB, semB)
            _copy40(dstsA, bidx, 4 * EB)
            _copy40(dstsB, bidx, 5 * EB)
            prow_n = prow0 + (gg * 2 + half + 1) * (6 * EB)
            pltpu.sync_copy(big_idx.at[pl.ds(prow_n, 6 * EB)], bidx_next)
            _softmax_batch(srcrowsA, dstrowsA, edgrowsA, dstsA, semA)
            _issue(bidx_next, pstart + 2 * EB, 0,
                   srcrowsA, dstrowsA, edgrowsA, semA)
            _softmax_batch(srcrowsB, dstrowsB, edgrowsB, dstsB, semB)

    pltpu.make_async_copy(src_tab.at[bidxA.at[pl.ds(0, EB)]],
                          srcrowsA, semA).wait()
    pltpu.make_async_copy(dst_tab.at[bidxA.at[pl.ds(2 * EB, EB)]],
                          dstrowsA, semA).wait()
    pltpu.make_async_copy(edg_tab.at[pl.ds(idx_off + ebase, EB)],
                          edgrowsA, semA).wait()

    wid = si * NC + ci
    nbase = wid * (N_EDGES // (NC * NS))

    def _ne_batch(rows_s, rows_d, start):
        pltpu.sync_copy(ee_tab.at[pl.ds(start, EB)], eerows)

        @pl.loop(0, EB)
        def _nbody(b):
            eerows[b, :] = (rows_s[b, pl.ds(0, EF)] +
                            rows_d[b, pl.ds(0, EF)] + eerows[b, :])

        pltpu.sync_copy(eerows, out_e.at[pl.ds(start, EB)])

    @pl.loop(0, N_EDGES // (NC * NS * EB * 2))
    def _nepair(g):
        start = nbase + g * (2 * EB)
        p0 = (nbase // (2 * EB) + g) * (6 * EB)
        pltpu.sync_copy(big_idx.at[pl.ds(p0, 6 * EB)], bidxA)
        cpA1 = pltpu.async_copy(xe_tab.at[bidxA.at[pl.ds(0, EB)]],
                                srcrowsA, semA)
        cpA2 = pltpu.async_copy(xe_tab.at[bidxA.at[pl.ds(4 * EB, EB)]],
                                dstrowsA, semA)
        cpB1 = pltpu.async_copy(xe_tab.at[bidxA.at[pl.ds(EB, EB)]],
                                srcrowsB, semB)
        cpB2 = pltpu.async_copy(xe_tab.at[bidxA.at[pl.ds(5 * EB, EB)]],
                                dstrowsB, semB)
        cpA1.wait()
        cpA2.wait()
        _ne_batch(srcrowsA, dstrowsA, start)
        cpB1.wait()
        cpB2.wait()
        _ne_batch(srcrowsB, dstrowsB, start + EB)

    plsc.subcore_barrier()

    eps = jnp.full((16,), 1e-16, jnp.float32)
    bts = [btvec[pl.ds(mbase + h * 16, 16)] for h in range(4)]
    for k in range(NODES_PER_TILE // EB):
        base = si * NODES_PER_TILE + k * EB
        pltpu.sync_copy(s_acc.at[pl.ds(base, EB)], srcrowsA)

        @pl.loop(0, EB)
        def _ebody(r):
            for h in range(4):
                co = h * 16
                s0 = srcrowsA[r, pl.ds(co, 16)]
                s1 = srcrowsA[r, pl.ds(co + H, 16)]
                ebo[r, pl.ds(co, 16)] = s1 / (s0 + eps) + bts[h]

        pltpu.sync_copy(ebo, out_x.at[pl.ds(ci * NPAD + base, EB)])


_sc_pass = functools.partial(
    pl.kernel,
    out_type=[
        jax.ShapeDtypeStruct((NC * NPAD, H), jnp.float32),
        jax.ShapeDtypeStruct((N_EDGES, EF), jnp.float32),
    ],
    mesh=plsc.VectorSubcoreMesh(
        core_axis_name="c", subcore_axis_name="s", num_cores=NC,
        num_subcores=NS),
    scratch_types=[
        pltpu.VMEM_SHARED((NPAD, D), jnp.float32),
        pltpu.VMEM((6 * EB,), jnp.int32),
        pltpu.VMEM((6 * EB,), jnp.int32),
        pltpu.VMEM((EB,), jnp.int32),
        pltpu.VMEM((EB,), jnp.int32),
        pltpu.VMEM((EB, D), jnp.float32),
        pltpu.VMEM((EB, D), jnp.float32),
        pltpu.VMEM((EB, D), jnp.float32),
        pltpu.VMEM((EB, D), jnp.float32),
        pltpu.VMEM((EB, D), jnp.float32),
        pltpu.VMEM((EB, D), jnp.float32),
        pltpu.VMEM((EB, D), jnp.float32),
        pltpu.VMEM((EB, EF), jnp.float32),
        pltpu.VMEM((EB, H), jnp.float32),
        pltpu.VMEM((D,), jnp.float32),
        pltpu.VMEM((D,), jnp.float32),
        pltpu.VMEM((16,), jnp.float32),
        pltpu.SemaphoreType.DMA,
        pltpu.SemaphoreType.DMA,
    ],
)(_sc_body)


def kernel(x, edge_index, edge_attr, W_a, W_T, b_T, W_e, W_ee, prelu_w):
    x = x.astype(jnp.float32)
    e = edge_attr.astype(jnp.float32)
    src = edge_index[0].astype(jnp.int32)
    dst = edge_index[1].astype(jnp.int32)

    A2, Ae, A1 = W_a[:V_IN], W_a[V_IN:V_IN + EF], W_a[V_IN + EF:]
    T2, Te, T1 = W_T[:V_IN], W_T[V_IN:V_IN + EF], W_T[V_IN + EF:]

    def halves(a_part, t_part):
        return jnp.stack([
            jnp.concatenate([a_part[:, :H], t_part[:, :H]], axis=1),
            jnp.concatenate([a_part[:, H:], t_part[:, H:]], axis=1),
        ])

    ws = halves(A1, T1)
    wd = halves(A2, T2)
    wa = halves(Ae, Te)

    wep = jnp.zeros((V_IN, D), jnp.float32).at[:, :EF].set(W_e)
    src_pair, dst_pair, xe = _node_tables(x, ws, wd, wep)
    edg_pair, ee, amx, amn = _edge_tables(e, wa, W_ee)

    smax = jnp.concatenate([src_pair[0, :, :H].max(0), src_pair[1, :, :H].max(0)])
    smin = jnp.concatenate([src_pair[0, :, :H].min(0), src_pair[1, :, :H].min(0)])
    dmax = jnp.concatenate([dst_pair[0, :, :H].max(0), dst_pair[1, :, :H].max(0)])
    dmin = jnp.concatenate([dst_pair[0, :, :H].min(0), dst_pair[1, :, :H].min(0)])
    emax = amx.max(axis=(0, 1))
    emin = amn.min(axis=(0, 1))
    hi = smax + dmax + emax
    lo = smin + dmin + emin
    mvec = jnp.maximum(hi, jnp.maximum(prelu_w * hi, prelu_w * lo))
    mvec = mvec.astype(jnp.float32)

    src_tab = src_pair.reshape(NC * N_NODES, D)
    dst_tab = dst_pair.reshape(NC * N_NODES, D)
    edg_tab = edg_pair.reshape(NC * N_EDGES, D)
    pwv = jnp.full((16,), prelu_w, jnp.float32)
    srcp = src.reshape(-1, 2 * EB)
    dstp = dst.reshape(-1, 2 * EB)
    big = jnp.concatenate(
        [jnp.concatenate([srcp + h2 * N_NODES, dstp + h2 * N_NODES, dstp],
                         axis=1) for h2 in range(NC)]).reshape(-1)
    big = jnp.concatenate([big, jnp.zeros((6 * EB,), jnp.int32)])

    out_x, out_e = _sc_pass(src_tab, dst_tab, edg_tab, xe, ee, big,
                            mvec, b_T.astype(jnp.float32), pwv)

    new_x = jnp.concatenate([out_x[:N_NODES], out_x[NPAD:NPAD + N_NODES]],
                            axis=1)
    return (new_x, out_e)


# speedup vs baseline: 1.9336x; 1.0101x over previous
import functools

import jax
import jax.numpy as jnp
from jax import lax
from jax.experimental import pallas as pl
from jax.experimental.pallas import tpu as pltpu
from jax.experimental.pallas import tpu_sc as plsc

N_NODES = 10000
N_EDGES = 320000
V_IN = 128
D = 128
EF = 16
H = 64
NC = 2
NS = 16
EB = 40
EDGES_PER_TILE = N_EDGES // NS
NBATCH = EDGES_PER_TILE // EB
NPAD = 10240
NODES_PER_TILE = NPAD // NS
EPI_CHUNK = 64
NODE_BLK = 2000
EDGE_BLK = 8000


def _node_tables_body(x_ref, ws_ref, wd_ref, we_ref, src_ref, dst_ref, xe_ref):
    xb = x_ref[...]
    src_ref[0] = jnp.dot(xb, ws_ref[0], preferred_element_type=jnp.float32)
    src_ref[1] = jnp.dot(xb, ws_ref[1], preferred_element_type=jnp.float32)
    dst_ref[0] = jnp.dot(xb, wd_ref[0], preferred_element_type=jnp.float32)
    dst_ref[1] = jnp.dot(xb, wd_ref[1], preferred_element_type=jnp.float32)
    xe_ref[...] = jnp.dot(xb, we_ref[...], preferred_element_type=jnp.float32)


def _node_tables(x, ws, wd, we):
    nblk = N_NODES // NODE_BLK
    return pl.pallas_call(
        _node_tables_body,
        grid=(nblk,),
        in_specs=[
            pl.BlockSpec((NODE_BLK, V_IN), lambda i: (i, 0)),
            pl.BlockSpec((NC, V_IN, D), lambda i: (0, 0, 0)),
            pl.BlockSpec((NC, V_IN, D), lambda i: (0, 0, 0)),
            pl.BlockSpec((V_IN, D), lambda i: (0, 0)),
        ],
        out_specs=[
            pl.BlockSpec((NC, NODE_BLK, D), lambda i: (0, i, 0)),
            pl.BlockSpec((NC, NODE_BLK, D), lambda i: (0, i, 0)),
            pl.BlockSpec((NODE_BLK, D), lambda i: (i, 0)),
        ],
        out_shape=[
            jax.ShapeDtypeStruct((NC, N_NODES, D), jnp.float32),
            jax.ShapeDtypeStruct((NC, N_NODES, D), jnp.float32),
            jax.ShapeDtypeStruct((N_NODES, D), jnp.float32),
        ],
    )(x, ws, wd, we)


def _edge_tables_body(e_ref, wa_ref, wee_ref, edg_ref, ee_ref, mx_ref, mn_ref):
    eb = e_ref[...]
    o0 = jnp.dot(eb, wa_ref[0], preferred_element_type=jnp.float32)
    o1 = jnp.dot(eb, wa_ref[1], preferred_element_type=jnp.float32)
    edg_ref[0] = o0
    edg_ref[1] = o1
    ee_ref[...] = jnp.dot(eb, wee_ref[...], preferred_element_type=jnp.float32)
    acat = jnp.concatenate([o0[:, :H], o1[:, :H]], axis=1)
    mx_ref[0] = jnp.broadcast_to(jnp.max(acat, axis=0, keepdims=True), (8, D))
    mn_ref[0] = jnp.broadcast_to(jnp.min(acat, axis=0, keepdims=True), (8, D))


def _edge_tables(e, wa, wee):
    nblk = N_EDGES // EDGE_BLK
    return pl.pallas_call(
        _edge_tables_body,
        grid=(nblk,),
        in_specs=[
            pl.BlockSpec((EDGE_BLK, EF), lambda i: (i, 0)),
            pl.BlockSpec((NC, EF, D), lambda i: (0, 0, 0)),
            pl.BlockSpec((EF, EF), lambda i: (0, 0)),
        ],
        out_specs=[
            pl.BlockSpec((NC, EDGE_BLK, D), lambda i: (0, i, 0)),
            pl.BlockSpec((EDGE_BLK, EF), lambda i: (i, 0)),
            pl.BlockSpec((1, 8, D), lambda i: (i, 0, 0)),
            pl.BlockSpec((1, 8, D), lambda i: (i, 0, 0)),
        ],
        out_shape=[
            jax.ShapeDtypeStruct((NC, N_EDGES, D), jnp.float32),
            jax.ShapeDtypeStruct((N_EDGES, EF), jnp.float32),
            jax.ShapeDtypeStruct((nblk, 8, D), jnp.float32),
            jax.ShapeDtypeStruct((nblk, 8, D), jnp.float32),
        ],
    )(e, wa, wee)


def _sc_body(src_tab, dst_tab, edg_tab, xe_tab, ee_tab,
             big_idx,
             m_hbm, bt_hbm, pw_hbm,
             out_x, out_e,
             s_acc,
             bidxA, bidxB, dstsA, dstsB,
             srcrowsA, dstrowsA, edgrowsA,
             srcrowsB, dstrowsB, edgrowsB,
             scat, scatB, eerows,
             mvec, btvec, pwvec,
             semA, semB, semS):
    ci = lax.axis_index("c")
    si = lax.axis_index("s")
    mbase = ci * H

    pltpu.sync_copy(m_hbm, mvec)
    pltpu.sync_copy(bt_hbm, btvec)
    pltpu.sync_copy(pw_hbm, pwvec)
    pwv = pwvec[...]
    zero16 = jnp.zeros((16,), jnp.float32)

    @pl.loop(0, EB * 8)
    def _zbody(i):
        r = lax.shift_right_logical(i, 3)
        co = jnp.bitwise_and(i, 7) * 16
        scat[r, pl.ds(co, 16)] = zero16

    for k in range(NODES_PER_TILE // EB):
        base = si * NODES_PER_TILE + k * EB
        pltpu.sync_copy(scat, s_acc.at[pl.ds(base, EB)])
    plsc.subcore_barrier()

    ebase = si * EDGES_PER_TILE
    idx_off = ci * N_EDGES
    mvs = [mvec[pl.ds(mbase + h * 16, 16)] for h in range(4)]

    def _copy40(dst_ref, src_ref, off):
        for c in (0, 16, 24):
            dst_ref[pl.ds(c, 16)] = src_ref[pl.ds(off + c, 16)]

    def _softmax_batch(rows_s, rows_d, rows_e, sbuf, sem):
        pltpu.make_async_copy(src_tab.at[bidxA.at[pl.ds(0, EB)]],
                              rows_s, sem).wait()
        pltpu.make_async_copy(src_tab.at[bidxA.at[pl.ds(0, EB)]],
                              rows_d, sem).wait()
        pltpu.make_async_copy(src_tab.at[bidxA.at[pl.ds(0, EB)]],
                              rows_e, sem).wait()

        @pl.loop(0, EB)
        def _cbody(b):
            for h in range(4):
                co = h * 16
                a1 = rows_s[b, pl.ds(co, 16)]
                a2 = rows_d[b, pl.ds(co, 16)]
                ae = rows_e[b, pl.ds(co, 16)]
                lin = a1 + a2 + ae
                logit = jnp.where(lin >= 0.0, lin, pwv * lin)
                ex = jnp.exp(logit - mvs[h])
                t1 = rows_s[b, pl.ds(co + H, 16)]
                t2 = rows_d[b, pl.ds(co + H, 16)]
                te = rows_e[b, pl.ds(co + H, 16)]
                sbuf[b, pl.ds(co, 16)] = ex
                sbuf[b, pl.ds(co + H, 16)] = ex * (t1 + t2 + te)

    NPAIR_T = NBATCH // 2
    prow0 = (ci * (N_EDGES // (2 * EB)) + si * NPAIR_T) * (6 * EB)

    def _issue(bidx, pstart, batch, rows_s, rows_d, rows_e, sem):
        cp1 = pltpu.async_copy(src_tab.at[bidx.at[pl.ds(batch * EB, EB)]],
                               rows_s, sem)
        cp2 = pltpu.async_copy(
            dst_tab.at[bidx.at[pl.ds((2 + batch) * EB, EB)]], rows_d, sem)
        cp3 = pltpu.async_copy(
            edg_tab.at[pl.ds(idx_off + pstart + batch * EB, EB)], rows_e, sem)
        return cp1, cp2, cp3

    pltpu.sync_copy(big_idx.at[pl.ds(prow0, 6 * EB)], bidxA)
    _issue(bidxA, ebase, 0, srcrowsA, dstrowsA, edgrowsA, semA)

    @pl.loop(0, NPAIR_T // 2)
    def _pairpair(gg):
        start0 = ebase + gg * (4 * EB)
        for half in range(2):
            bidx = (bidxA, bidxB)[half]
            bidx_next = (bidxB, bidxA)[half]
            pstart = start0 + half * (2 * EB)
            _issue(bidx, pstart, 1, srcrowsB, dstrowsB, edgrowsB, semB)
            _copy40(dstsA, bidx, 4 * EB)
            _copy40(dstsB, bidx, 5 * EB)
            prow_n = prow0 + (gg * 2 + half + 1) * (6 * EB)
            pltpu.sync_copy(big_idx.at[pl.ds(prow_n, 6 * EB)], bidx_next)
            _softmax_batch(srcrowsA, dstrowsA, edgrowsA, scat, semA)
            scA = pltpu.async_copy(scat, s_acc.at[dstsA], semS, add=True)
            _issue(bidx_next, pstart + 2 * EB, 0,
                   srcrowsA, dstrowsA, edgrowsA, semA)
            _softmax_batch(srcrowsB, dstrowsB, edgrowsB, scatB, semB)
            scA.wait()
            pltpu.sync_copy(scatB, s_acc.at[dstsB], add=True)

    pltpu.make_async_copy(src_tab.at[bidxA.at[pl.ds(0, EB)]],
                          srcrowsA, semA).wait()
    pltpu.make_async_copy(dst_tab.at[bidxA.at[pl.ds(2 * EB, EB)]],
                          dstrowsA, semA).wait()
    pltpu.make_async_copy(edg_tab.at[pl.ds(idx_off + ebase, EB)],
                          edgrowsA, semA).wait()

    wid = si * NC + ci
    nbase = wid * (N_EDGES // (NC * NS))

    def _ne_batch(rows_s, rows_d, start):
        pltpu.sync_copy(ee_tab.at[pl.ds(start, EB)], eerows)

        @pl.loop(0, EB)
        def _nbody(b):
            eerows[b, :] = (rows_s[b, pl.ds(0, EF)] +
                            rows_d[b, pl.ds(0, EF)] + eerows[b, :])

        pltpu.sync_copy(eerows, out_e.at[pl.ds(start, EB)])

    @pl.loop(0, N_EDGES // (NC * NS * EB * 2))
    def _nepair(g):
        start = nbase + g * (2 * EB)
        p0 = (nbase // (2 * EB) + g) * (6 * EB)
        pltpu.sync_copy(big_idx.at[pl.ds(p0, 6 * EB)], bidxA)
        cpA1 = pltpu.async_copy(xe_tab.at[bidxA.at[pl.ds(0, EB)]],
                                srcrowsA, semA)
        cpA2 = pltpu.async_copy(xe_tab.at[bidxA.at[pl.ds(4 * EB, EB)]],
                                dstrowsA, semA)
        cpB1 = pltpu.async_copy(xe_tab.at[bidxA.at[pl.ds(EB, EB)]],
                                srcrowsB, semB)
        cpB2 = pltpu.async_copy(xe_tab.at[bidxA.at[pl.ds(5 * EB, EB)]],
                                dstrowsB, semB)
        cpA1.wait()
        cpA2.wait()
        _ne_batch(srcrowsA, dstrowsA, start)
        cpB1.wait()
        cpB2.wait()
        _ne_batch(srcrowsB, dstrowsB, start + EB)

    plsc.subcore_barrier()

    eps = jnp.full((16,), 1e-16, jnp.float32)
    bts = [btvec[pl.ds(mbase + h * 16, 16)] for h in range(4)]
    for k in range(NODES_PER_TILE // EB):
        base = si * NODES_PER_TILE + k * EB
        pltpu.sync_copy(s_acc.at[pl.ds(base, EB)], srcrowsA)

        @pl.loop(0, EB)
        def _ebody(r):
            for h in range(4):
                co = h * 16
                s0 = srcrowsA[r, pl.ds(co, 16)]
                s1 = srcrowsA[r, pl.ds(co + H, 16)]
                scat[r, pl.ds(co, 16)] = s1 / (s0 + eps) + bts[h]
                scat[r, pl.ds(co + H, 16)] = s0

        pltpu.sync_copy(scat, out_x.at[pl.ds(ci * NPAD + base, EB)])


_sc_pass = functools.partial(
    pl.kernel,
    out_type=[
        jax.ShapeDtypeStruct((NC * NPAD, D), jnp.float32),
        jax.ShapeDtypeStruct((N_EDGES, EF), jnp.float32),
    ],
    mesh=plsc.VectorSubcoreMesh(
        core_axis_name="c", subcore_axis_name="s", num_cores=NC,
        num_subcores=NS),
    scratch_types=[
        pltpu.VMEM_SHARED((NPAD, D), jnp.float32),
        pltpu.VMEM((6 * EB,), jnp.int32),
        pltpu.VMEM((6 * EB,), jnp.int32),
        pltpu.VMEM((EB,), jnp.int32),
        pltpu.VMEM((EB,), jnp.int32),
        pltpu.VMEM((EB, D), jnp.float32),
        pltpu.VMEM((EB, D), jnp.float32),
        pltpu.VMEM((EB, D), jnp.float32),
        pltpu.VMEM((EB, D), jnp.float32),
        pltpu.VMEM((EB, D), jnp.float32),
        pltpu.VMEM((EB, D), jnp.float32),
        pltpu.VMEM((EB, D), jnp.float32),
        pltpu.VMEM((EB, D), jnp.float32),
        pltpu.VMEM((EB, EF), jnp.float32),
        pltpu.VMEM((D,), jnp.float32),
        pltpu.VMEM((D,), jnp.float32),
        pltpu.VMEM((16,), jnp.float32),
        pltpu.SemaphoreType.DMA,
        pltpu.SemaphoreType.DMA,
        pltpu.SemaphoreType.DMA,
    ],
)(_sc_body)


def kernel(x, edge_index, edge_attr, W_a, W_T, b_T, W_e, W_ee, prelu_w):
    x = x.astype(jnp.float32)
    e = edge_attr.astype(jnp.float32)
    src = edge_index[0].astype(jnp.int32)
    dst = edge_index[1].astype(jnp.int32)

    A2, Ae, A1 = W_a[:V_IN], W_a[V_IN:V_IN + EF], W_a[V_IN + EF:]
    T2, Te, T1 = W_T[:V_IN], W_T[V_IN:V_IN + EF], W_T[V_IN + EF:]

    def halves(a_part, t_part):
        return jnp.stack([
            jnp.concatenate([a_part[:, :H], t_part[:, :H]], axis=1),
            jnp.concatenate([a_part[:, H:], t_part[:, H:]], axis=1),
        ])

    ws = halves(A1, T1)
    wd = halves(A2, T2)
    wa = halves(Ae, Te)

    wep = jnp.zeros((V_IN, D), jnp.float32).at[:, :EF].set(W_e)
    src_pair, dst_pair, xe = _node_tables(x, ws, wd, wep)
    edg_pair, ee, amx, amn = _edge_tables(e, wa, W_ee)

    smax = jnp.concatenate([src_pair[0, :, :H].max(0), src_pair[1, :, :H].max(0)])
    smin = jnp.concatenate([src_pair[0, :, :H].min(0), src_pair[1, :, :H].min(0)])
    dmax = jnp.concatenate([dst_pair[0, :, :H].max(0), dst_pair[1, :, :H].max(0)])
    dmin = jnp.concatenate([dst_pair[0, :, :H].min(0), dst_pair[1, :, :H].min(0)])
    emax = amx.max(axis=(0, 1))
    emin = amn.min(axis=(0, 1))
    hi = smax + dmax + emax
    lo = smin + dmin + emin
    mvec = jnp.maximum(hi, jnp.maximum(prelu_w * hi, prelu_w * lo))
    mvec = mvec.astype(jnp.float32)

    src_tab = src_pair.reshape(NC * N_NODES, D)
    dst_tab = dst_pair.reshape(NC * N_NODES, D)
    edg_tab = edg_pair.reshape(NC * N_EDGES, D)
    pwv = jnp.full((16,), prelu_w, jnp.float32)
    srcp = src.reshape(-1, 2 * EB)
    dstp = dst.reshape(-1, 2 * EB)
    big = jnp.concatenate(
        [jnp.concatenate([srcp + h2 * N_NODES, dstp + h2 * N_NODES, dstp],
                         axis=1) for h2 in range(NC)]).reshape(-1)
    big = jnp.concatenate([big, jnp.zeros((6 * EB,), jnp.int32)])

    out_x, out_e = _sc_pass(src_tab, dst_tab, edg_tab, xe, ee, big,
                            mvec, b_T.astype(jnp.float32), pwv)

    new_x = jnp.concatenate([out_x[:N_NODES, :H], out_x[NPAD:NPAD + N_NODES, :H]],
                            axis=1)
    return (new_x, out_e)


# speedup vs baseline: 2.1207x; 1.0967x over previous
import functools

import jax
import jax.numpy as jnp
from jax import lax
from jax.experimental import pallas as pl
from jax.experimental.pallas import tpu as pltpu
from jax.experimental.pallas import tpu_sc as plsc

N_NODES = 10000
N_EDGES = 320000
V_IN = 128
D = 128
EF = 16
H = 64
NC = 2
NS = 16
EB = 40
EDGES_PER_TILE = N_EDGES // NS
NBATCH = EDGES_PER_TILE // EB
NPAD = 10240
NODES_PER_TILE = NPAD // NS
EPI_CHUNK = 64
NODE_BLK = 2000
EDGE_BLK = 8000


def _node_tables_body(x_ref, ws_ref, wd_ref, we_ref, src_ref, dst_ref, xe_ref):
    xb = x_ref[...]
    src_ref[0] = jnp.dot(xb, ws_ref[0], preferred_element_type=jnp.float32)
    src_ref[1] = jnp.dot(xb, ws_ref[1], preferred_element_type=jnp.float32)
    dst_ref[0] = jnp.dot(xb, wd_ref[0], preferred_element_type=jnp.float32)
    dst_ref[1] = jnp.dot(xb, wd_ref[1], preferred_element_type=jnp.float32)
    xe_ref[...] = jnp.dot(xb, we_ref[...], preferred_element_type=jnp.float32)


def _node_tables(x, ws, wd, we):
    nblk = N_NODES // NODE_BLK
    return pl.pallas_call(
        _node_tables_body,
        grid=(nblk,),
        in_specs=[
            pl.BlockSpec((NODE_BLK, V_IN), lambda i: (i, 0)),
            pl.BlockSpec((NC, V_IN, D), lambda i: (0, 0, 0)),
            pl.BlockSpec((NC, V_IN, D), lambda i: (0, 0, 0)),
            pl.BlockSpec((V_IN, D), lambda i: (0, 0)),
        ],
        out_specs=[
            pl.BlockSpec((NC, NODE_BLK, D), lambda i: (0, i, 0)),
            pl.BlockSpec((NC, NODE_BLK, D), lambda i: (0, i, 0)),
            pl.BlockSpec((NODE_BLK, D), lambda i: (i, 0)),
        ],
        out_shape=[
            jax.ShapeDtypeStruct((NC, N_NODES, D), jnp.float32),
            jax.ShapeDtypeStruct((NC, N_NODES, D), jnp.float32),
            jax.ShapeDtypeStruct((N_NODES, D), jnp.float32),
        ],
    )(x, ws, wd, we)


def _edge_tables_body(e_ref, wa_ref, wee_ref, edg_ref, ee_ref, mx_ref, mn_ref):
    eb = e_ref[...]
    o0 = jnp.dot(eb, wa_ref[0], preferred_element_type=jnp.float32)
    o1 = jnp.dot(eb, wa_ref[1], preferred_element_type=jnp.float32)
    edg_ref[0] = o0
    edg_ref[1] = o1
    ee_ref[...] = jnp.dot(eb, wee_ref[...], preferred_element_type=jnp.float32)
    acat = jnp.concatenate([o0[:, :H], o1[:, :H]], axis=1)
    mx_ref[0] = jnp.broadcast_to(jnp.max(acat, axis=0, keepdims=True), (8, D))
    mn_ref[0] = jnp.broadcast_to(jnp.min(acat, axis=0, keepdims=True), (8, D))


def _edge_tables(e, wa, wee):
    nblk = N_EDGES // EDGE_BLK
    return pl.pallas_call(
        _edge_tables_body,
        grid=(nblk,),
        in_specs=[
            pl.BlockSpec((EDGE_BLK, EF), lambda i: (i, 0)),
            pl.BlockSpec((NC, EF, D), lambda i: (0, 0, 0)),
            pl.BlockSpec((EF, EF), lambda i: (0, 0)),
        ],
        out_specs=[
            pl.BlockSpec((NC, EDGE_BLK, D), lambda i: (0, i, 0)),
            pl.BlockSpec((EDGE_BLK, EF), lambda i: (i, 0)),
            pl.BlockSpec((1, 8, D), lambda i: (i, 0, 0)),
            pl.BlockSpec((1, 8, D), lambda i: (i, 0, 0)),
        ],
        out_shape=[
            jax.ShapeDtypeStruct((NC, N_EDGES, D), jnp.float32),
            jax.ShapeDtypeStruct((N_EDGES, EF), jnp.float32),
            jax.ShapeDtypeStruct((nblk, 8, D), jnp.float32),
            jax.ShapeDtypeStruct((nblk, 8, D), jnp.float32),
        ],
    )(e, wa, wee)


def _sc_body(src_tab, dst_tab, edg_tab, xe_tab, ee_tab,
             big_idx,
             m_hbm, bt_hbm, pw_hbm,
             out_x, out_e,
             s_acc,
             bidxA, bidxB, dstsA, dstsB,
             srcrowsA, dstrowsA, edgrowsA,
             srcrowsB, dstrowsB, edgrowsB,
             scat, scatB, eerows,
             mvec, btvec, pwvec,
             semA, semB, semS):
    ci = lax.axis_index("c")
    si = lax.axis_index("s")
    mbase = ci * H

    pltpu.sync_copy(m_hbm, mvec)
    pltpu.sync_copy(bt_hbm, btvec)
    pltpu.sync_copy(pw_hbm, pwvec)
    pwv = pwvec[...]
    zero16 = jnp.zeros((16,), jnp.float32)

    @pl.loop(0, EB * 8)
    def _zbody(i):
        r = lax.shift_right_logical(i, 3)
        co = jnp.bitwise_and(i, 7) * 16
        scat[r, pl.ds(co, 16)] = zero16

    for k in range(NODES_PER_TILE // EB):
        base = si * NODES_PER_TILE + k * EB
        pltpu.sync_copy(scat, s_acc.at[pl.ds(base, EB)])
    plsc.subcore_barrier()

    ebase = si * EDGES_PER_TILE
    idx_off = ci * N_EDGES
    mvs = [mvec[pl.ds(mbase + h * 16, 16)] for h in range(4)]

    def _copy40(dst_ref, src_ref, off):
        for c in (0, 16, 24):
            dst_ref[pl.ds(c, 16)] = src_ref[pl.ds(off + c, 16)]

    def _softmax_batch(rows_s, rows_d, rows_e, sbuf, sem):
        pltpu.make_async_copy(src_tab.at[bidxA.at[pl.ds(0, EB)]],
                              rows_s, sem).wait()
        pltpu.make_async_copy(src_tab.at[bidxA.at[pl.ds(0, EB)]],
                              rows_d, sem).wait()
        pltpu.make_async_copy(src_tab.at[bidxA.at[pl.ds(0, EB)]],
                              rows_e, sem).wait()

        @pl.loop(0, EB)
        def _cbody(b):
            for h in range(4):
                co = h * 16
                a1 = rows_s[b, pl.ds(co, 16)]
                a2 = rows_d[b, pl.ds(co, 16)]
                ae = rows_e[b, pl.ds(co, 16)]
                lin = a1 + a2 + ae
                logit = jnp.where(lin >= 0.0, lin, pwv * lin)
                ex = jnp.exp(logit - mvs[h])
                t1 = rows_s[b, pl.ds(co + H, 16)]
                t2 = rows_d[b, pl.ds(co + H, 16)]
                te = rows_e[b, pl.ds(co + H, 16)]
                sbuf[b, pl.ds(co, 16)] = ex
                sbuf[b, pl.ds(co + H, 16)] = ex * (t1 + t2 + te)

    NPAIR_T = NBATCH // 2
    prow0 = (ci * (N_EDGES // (2 * EB)) + si * NPAIR_T) * (6 * EB)

    def _issue(bidx, pstart, batch, rows_s, rows_d, rows_e, sem):
        cp1 = pltpu.async_copy(src_tab.at[bidx.at[pl.ds(batch * EB, EB)]],
                               rows_s, sem)
        cp2 = pltpu.async_copy(
            dst_tab.at[bidx.at[pl.ds((2 + batch) * EB, EB)]], rows_d, sem)
        cp3 = pltpu.async_copy(
            edg_tab.at[pl.ds(idx_off + pstart + batch * EB, EB)], rows_e, sem)
        return cp1, cp2, cp3

    pltpu.sync_copy(big_idx.at[pl.ds(prow0, 6 * EB)], bidxA)
    _issue(bidxA, ebase, 0, srcrowsA, dstrowsA, edgrowsA, semA)

    @pl.loop(0, NPAIR_T // 2)
    def _pairpair(gg):
        start0 = ebase + gg * (4 * EB)
        for half in range(2):
            bidx = (bidxA, bidxB)[half]
            bidx_next = (bidxB, bidxA)[half]
            pstart = start0 + half * (2 * EB)
            _issue(bidx, pstart, 1, srcrowsB, dstrowsB, edgrowsB, semB)
            _copy40(dstsA, bidx, 4 * EB)
            _copy40(dstsB, bidx, 5 * EB)
            prow_n = prow0 + (gg * 2 + half + 1) * (6 * EB)
            pltpu.sync_copy(big_idx.at[pl.ds(prow_n, 6 * EB)], bidx_next)
            _softmax_batch(srcrowsA, dstrowsA, edgrowsA, scat, semA)
            scA = pltpu.async_copy(scat, s_acc.at[dstsA], semS, add=True)
            _issue(bidx_next, pstart + 2 * EB, 0,
                   srcrowsA, dstrowsA, edgrowsA, semA)
            _softmax_batch(srcrowsB, dstrowsB, edgrowsB, scatB, semB)
            scA.wait()
            pltpu.sync_copy(scatB, s_acc.at[dstsB], add=True)

    pltpu.make_async_copy(src_tab.at[bidxA.at[pl.ds(0, EB)]],
                          srcrowsA, semA).wait()
    pltpu.make_async_copy(dst_tab.at[bidxA.at[pl.ds(2 * EB, EB)]],
                          dstrowsA, semA).wait()
    pltpu.make_async_copy(edg_tab.at[pl.ds(idx_off + ebase, EB)],
                          edgrowsA, semA).wait()

    wid = si * NC + ci
    nbase = wid * (N_EDGES // (NC * NS))
    ne_row0 = nbase // (2 * EB)

    def _ne_issue(bidx, batch, rows_s, rows_d, sem):
        pltpu.async_copy(xe_tab.at[bidx.at[pl.ds(batch * EB, EB)]],
                         rows_s, sem)
        pltpu.async_copy(xe_tab.at[bidx.at[pl.ds((4 + batch) * EB, EB)]],
                         rows_d, sem)

    def _ne_drain(rows_s, rows_d, sem):
        pltpu.make_async_copy(xe_tab.at[bidxA.at[pl.ds(0, EB)]],
                              rows_s, sem).wait()
        pltpu.make_async_copy(xe_tab.at[bidxA.at[pl.ds(0, EB)]],
                              rows_d, sem).wait()

    def _ne_proc(rows_s, rows_d, start):
        pltpu.sync_copy(ee_tab.at[pl.ds(start, EB)], eerows)

        @pl.loop(0, EB)
        def _nbody(b):
            eerows[b, :] = (rows_s[b, pl.ds(0, EF)] +
                            rows_d[b, pl.ds(0, EF)] + eerows[b, :])

        pltpu.sync_copy(eerows, out_e.at[pl.ds(start, EB)])

    pltpu.sync_copy(big_idx.at[pl.ds(ne_row0 * (6 * EB), 6 * EB)], bidxA)
    _ne_issue(bidxA, 0, srcrowsA, dstrowsA, semA)

    @pl.loop(0, N_EDGES // (NC * NS * EB * 2))
    def _nepair(g):
        start = nbase + g * (2 * EB)
        _ne_issue(bidxA, 1, srcrowsB, dstrowsB, semB)
        prow_n = (ne_row0 + g + 1) * (6 * EB)
        pltpu.sync_copy(big_idx.at[pl.ds(prow_n, 6 * EB)], bidxB)
        _ne_drain(srcrowsA, dstrowsA, semA)
        _ne_proc(srcrowsA, dstrowsA, start)
        _ne_issue(bidxB, 0, srcrowsA, dstrowsA, semA)
        _ne_drain(srcrowsB, dstrowsB, semB)
        _ne_proc(srcrowsB, dstrowsB, start + EB)
        for c in range(0, 6 * EB, 16):
            bidxA[pl.ds(c, 16)] = bidxB[pl.ds(c, 16)]

    _ne_drain(srcrowsA, dstrowsA, semA)

    plsc.subcore_barrier()

    eps = jnp.full((16,), 1e-16, jnp.float32)
    bts = [btvec[pl.ds(mbase + h * 16, 16)] for h in range(4)]
    for k in range(NODES_PER_TILE // EB):
        base = si * NODES_PER_TILE + k * EB
        pltpu.sync_copy(s_acc.at[pl.ds(base, EB)], srcrowsA)

        @pl.loop(0, EB)
        def _ebody(r):
            for h in range(4):
                co = h * 16
                s0 = srcrowsA[r, pl.ds(co, 16)]
                s1 = srcrowsA[r, pl.ds(co + H, 16)]
                scat[r, pl.ds(co, 16)] = s1 / (s0 + eps) + bts[h]
                scat[r, pl.ds(co + H, 16)] = s0

        pltpu.sync_copy(scat, out_x.at[pl.ds(ci * NPAD + base, EB)])


_sc_pass = functools.partial(
    pl.kernel,
    out_type=[
        jax.ShapeDtypeStruct((NC * NPAD, D), jnp.float32),
        jax.ShapeDtypeStruct((N_EDGES, EF), jnp.float32),
    ],
    mesh=plsc.VectorSubcoreMesh(
        core_axis_name="c", subcore_axis_name="s", num_cores=NC,
        num_subcores=NS),
    scratch_types=[
        pltpu.VMEM_SHARED((NPAD, D), jnp.float32),
        pltpu.VMEM((6 * EB,), jnp.int32),
        pltpu.VMEM((6 * EB,), jnp.int32),
        pltpu.VMEM((EB,), jnp.int32),
        pltpu.VMEM((EB,), jnp.int32),
        pltpu.VMEM((EB, D), jnp.float32),
        pltpu.VMEM((EB, D), jnp.float32),
        pltpu.VMEM((EB, D), jnp.float32),
        pltpu.VMEM((EB, D), jnp.float32),
        pltpu.VMEM((EB, D), jnp.float32),
        pltpu.VMEM((EB, D), jnp.float32),
        pltpu.VMEM((EB, D), jnp.float32),
        pltpu.VMEM((EB, D), jnp.float32),
        pltpu.VMEM((EB, EF), jnp.float32),
        pltpu.VMEM((D,), jnp.float32),
        pltpu.VMEM((D,), jnp.float32),
        pltpu.VMEM((16,), jnp.float32),
        pltpu.SemaphoreType.DMA,
        pltpu.SemaphoreType.DMA,
        pltpu.SemaphoreType.DMA,
    ],
)(_sc_body)


def kernel(x, edge_index, edge_attr, W_a, W_T, b_T, W_e, W_ee, prelu_w):
    x = x.astype(jnp.float32)
    e = edge_attr.astype(jnp.float32)
    src = edge_index[0].astype(jnp.int32)
    dst = edge_index[1].astype(jnp.int32)

    A2, Ae, A1 = W_a[:V_IN], W_a[V_IN:V_IN + EF], W_a[V_IN + EF:]
    T2, Te, T1 = W_T[:V_IN], W_T[V_IN:V_IN + EF], W_T[V_IN + EF:]

    def halves(a_part, t_part):
        return jnp.stack([
            jnp.concatenate([a_part[:, :H], t_part[:, :H]], axis=1),
            jnp.concatenate([a_part[:, H:], t_part[:, H:]], axis=1),
        ])

    ws = halves(A1, T1)
    wd = halves(A2, T2)
    wa = halves(Ae, Te)

    wep = jnp.zeros((V_IN, D), jnp.float32).at[:, :EF].set(W_e)
    src_pair, dst_pair, xe = _node_tables(x, ws, wd, wep)
    edg_pair, ee, amx, amn = _edge_tables(e, wa, W_ee)

    smax = jnp.concatenate([src_pair[0, :, :H].max(0), src_pair[1, :, :H].max(0)])
    smin = jnp.concatenate([src_pair[0, :, :H].min(0), src_pair[1, :, :H].min(0)])
    dmax = jnp.concatenate([dst_pair[0, :, :H].max(0), dst_pair[1, :, :H].max(0)])
    dmin = jnp.concatenate([dst_pair[0, :, :H].min(0), dst_pair[1, :, :H].min(0)])
    emax = amx.max(axis=(0, 1))
    emin = amn.min(axis=(0, 1))
    hi = smax + dmax + emax
    lo = smin + dmin + emin
    mvec = jnp.maximum(hi, jnp.maximum(prelu_w * hi, prelu_w * lo))
    mvec = mvec.astype(jnp.float32)

    src_tab = src_pair.reshape(NC * N_NODES, D)
    dst_tab = dst_pair.reshape(NC * N_NODES, D)
    edg_tab = edg_pair.reshape(NC * N_EDGES, D)
    pwv = jnp.full((16,), prelu_w, jnp.float32)
    srcp = src.reshape(-1, 2 * EB)
    dstp = dst.reshape(-1, 2 * EB)
    big = jnp.concatenate(
        [jnp.concatenate([srcp + h2 * N_NODES, dstp + h2 * N_NODES, dstp],
                         axis=1) for h2 in range(NC)]).reshape(-1)
    big = jnp.concatenate([big, jnp.zeros((6 * EB,), jnp.int32)])

    out_x, out_e = _sc_pass(src_tab, dst_tab, edg_tab, xe, ee, big,
                            mvec, b_T.astype(jnp.float32), pwv)

    new_x = jnp.concatenate([out_x[:N_NODES, :H], out_x[NPAD:NPAD + N_NODES, :H]],
                            axis=1)
    return (new_x, out_e)


# speedup vs baseline: 2.1925x; 1.0339x over previous
import functools

import jax
import jax.numpy as jnp
from jax import lax
from jax.experimental import pallas as pl
from jax.experimental.pallas import tpu as pltpu
from jax.experimental.pallas import tpu_sc as plsc

N_NODES = 10000
N_EDGES = 320000
V_IN = 128
D = 128
EF = 16
H = 64
NC = 2
NS = 16
EB = 40
EDGES_PER_TILE = N_EDGES // NS
NBATCH = EDGES_PER_TILE // EB
NPAD = 10240
NODES_PER_TILE = NPAD // NS
EPI_CHUNK = 64
NODE_BLK = 2000
EDGE_BLK = 8000


def _node_tables_body(x_ref, ws_ref, wd_ref, we_ref, src_ref, dst_ref, xe_ref):
    xb = x_ref[...]
    src_ref[0] = jnp.dot(xb, ws_ref[0], preferred_element_type=jnp.float32)
    src_ref[1] = jnp.dot(xb, ws_ref[1], preferred_element_type=jnp.float32)
    dst_ref[0] = jnp.dot(xb, wd_ref[0], preferred_element_type=jnp.float32)
    dst_ref[1] = jnp.dot(xb, wd_ref[1], preferred_element_type=jnp.float32)
    xe_ref[...] = jnp.dot(xb, we_ref[...], preferred_element_type=jnp.float32)


def _node_tables(x, ws, wd, we):
    nblk = N_NODES // NODE_BLK
    return pl.pallas_call(
        _node_tables_body,
        grid=(nblk,),
        in_specs=[
            pl.BlockSpec((NODE_BLK, V_IN), lambda i: (i, 0)),
            pl.BlockSpec((NC, V_IN, D), lambda i: (0, 0, 0)),
            pl.BlockSpec((NC, V_IN, D), lambda i: (0, 0, 0)),
            pl.BlockSpec((V_IN, D), lambda i: (0, 0)),
        ],
        out_specs=[
            pl.BlockSpec((NC, NODE_BLK, D), lambda i: (0, i, 0)),
            pl.BlockSpec((NC, NODE_BLK, D), lambda i: (0, i, 0)),
            pl.BlockSpec((NODE_BLK, D), lambda i: (i, 0)),
        ],
        out_shape=[
            jax.ShapeDtypeStruct((NC, N_NODES, D), jnp.float32),
            jax.ShapeDtypeStruct((NC, N_NODES, D), jnp.float32),
            jax.ShapeDtypeStruct((N_NODES, D), jnp.float32),
        ],
    )(x, ws, wd, we)


def _edge_tables_body(e_ref, wa_ref, wee_ref, edg_ref, ee_ref, mx_ref, mn_ref):
    eb = e_ref[...]
    o0 = jnp.dot(eb, wa_ref[0], preferred_element_type=jnp.float32)
    o1 = jnp.dot(eb, wa_ref[1], preferred_element_type=jnp.float32)
    edg_ref[0] = o0
    edg_ref[1] = o1
    ee_ref[...] = jnp.dot(eb, wee_ref[...], preferred_element_type=jnp.float32)
    acat = jnp.concatenate([o0[:, :H], o1[:, :H]], axis=1)
    mx_ref[0] = jnp.broadcast_to(jnp.max(acat, axis=0, keepdims=True), (8, D))
    mn_ref[0] = jnp.broadcast_to(jnp.min(acat, axis=0, keepdims=True), (8, D))


def _edge_tables(e, wa, wee):
    nblk = N_EDGES // EDGE_BLK
    return pl.pallas_call(
        _edge_tables_body,
        grid=(nblk,),
        in_specs=[
            pl.BlockSpec((EDGE_BLK, EF), lambda i: (i, 0)),
            pl.BlockSpec((NC, EF, D), lambda i: (0, 0, 0)),
            pl.BlockSpec((EF, EF), lambda i: (0, 0)),
        ],
        out_specs=[
            pl.BlockSpec((NC, EDGE_BLK, D), lambda i: (0, i, 0)),
            pl.BlockSpec((EDGE_BLK, EF), lambda i: (i, 0)),
            pl.BlockSpec((1, 8, D), lambda i: (i, 0, 0)),
            pl.BlockSpec((1, 8, D), lambda i: (i, 0, 0)),
        ],
        out_shape=[
            jax.ShapeDtypeStruct((NC, N_EDGES, D), jnp.float32),
            jax.ShapeDtypeStruct((N_EDGES, EF), jnp.float32),
            jax.ShapeDtypeStruct((nblk, 8, D), jnp.float32),
            jax.ShapeDtypeStruct((nblk, 8, D), jnp.float32),
        ],
    )(e, wa, wee)


def _sc_body(src_tab, dst_tab, edg_tab, xe_tab, ee_tab,
             big_idx,
             m_hbm, bt_hbm, pw_hbm,
             out_x, out_e,
             s_acc,
             bidxA, bidxB, dstsA, dstsB, dstsC, dstsD,
             srcrowsA, dstrowsA, edgrowsA,
             srcrowsB, dstrowsB, edgrowsB,
             scat, scatB, eerows,
             mvec, btvec, pwvec,
             semA, semB, semS, semS2):
    ci = lax.axis_index("c")
    si = lax.axis_index("s")
    mbase = ci * H

    pltpu.sync_copy(m_hbm, mvec)
    pltpu.sync_copy(bt_hbm, btvec)
    pltpu.sync_copy(pw_hbm, pwvec)
    pwv = pwvec[...]
    zero16 = jnp.zeros((16,), jnp.float32)

    @pl.loop(0, EB * 8)
    def _zbody(i):
        r = lax.shift_right_logical(i, 3)
        co = jnp.bitwise_and(i, 7) * 16
        scat[r, pl.ds(co, 16)] = zero16

    for k in range(NODES_PER_TILE // EB):
        base = si * NODES_PER_TILE + k * EB
        pltpu.sync_copy(scat, s_acc.at[pl.ds(base, EB)])
    plsc.subcore_barrier()

    ebase = si * EDGES_PER_TILE
    idx_off = ci * N_EDGES
    mvs = [mvec[pl.ds(mbase + h * 16, 16)] for h in range(4)]

    def _copy40(dst_ref, src_ref, off):
        for c in (0, 16, 24):
            dst_ref[pl.ds(c, 16)] = src_ref[pl.ds(off + c, 16)]

    def _softmax_batch(rows_s, rows_d, rows_e, sbuf, sem):
        pltpu.make_async_copy(src_tab.at[bidxA.at[pl.ds(0, EB)]],
                              rows_s, sem).wait()
        pltpu.make_async_copy(src_tab.at[bidxA.at[pl.ds(0, EB)]],
                              rows_d, sem).wait()
        pltpu.make_async_copy(src_tab.at[bidxA.at[pl.ds(0, EB)]],
                              rows_e, sem).wait()

        @pl.loop(0, EB)
        def _cbody(b):
            for h in range(4):
                co = h * 16
                a1 = rows_s[b, pl.ds(co, 16)]
                a2 = rows_d[b, pl.ds(co, 16)]
                ae = rows_e[b, pl.ds(co, 16)]
                lin = a1 + a2 + ae
                logit = jnp.where(lin >= 0.0, lin, pwv * lin)
                ex = jnp.exp(logit - mvs[h])
                t1 = rows_s[b, pl.ds(co + H, 16)]
                t2 = rows_d[b, pl.ds(co + H, 16)]
                te = rows_e[b, pl.ds(co + H, 16)]
                sbuf[b, pl.ds(co, 16)] = ex
                sbuf[b, pl.ds(co + H, 16)] = ex * (t1 + t2 + te)

    NPAIR_T = NBATCH // 2
    prow0 = (ci * (N_EDGES // (2 * EB)) + si * NPAIR_T) * (6 * EB)

    def _issue(bidx, pstart, batch, rows_s, rows_d, rows_e, sem):
        cp1 = pltpu.async_copy(src_tab.at[bidx.at[pl.ds(batch * EB, EB)]],
                               rows_s, sem)
        cp2 = pltpu.async_copy(
            dst_tab.at[bidx.at[pl.ds((2 + batch) * EB, EB)]], rows_d, sem)
        cp3 = pltpu.async_copy(
            edg_tab.at[pl.ds(idx_off + pstart + batch * EB, EB)], rows_e, sem)
        return cp1, cp2, cp3

    pltpu.async_copy(edg_tab.at[pl.ds(idx_off, EB)], scatB, semS2)
    pltpu.sync_copy(big_idx.at[pl.ds(prow0, 6 * EB)], bidxA)
    _issue(bidxA, ebase, 0, srcrowsA, dstrowsA, edgrowsA, semA)

    @pl.loop(0, NPAIR_T // 2)
    def _pairpair(gg):
        start0 = ebase + gg * (4 * EB)
        for half in range(2):
            bidx = (bidxA, bidxB)[half]
            bidx_next = (bidxB, bidxA)[half]
            d0 = (dstsA, dstsC)[half]
            d1 = (dstsB, dstsD)[half]
            pstart = start0 + half * (2 * EB)
            _issue(bidx, pstart, 1, srcrowsB, dstrowsB, edgrowsB, semB)
            _copy40(d0, bidx, 4 * EB)
            _copy40(d1, bidx, 5 * EB)
            prow_n = prow0 + (gg * 2 + half + 1) * (6 * EB)
            pltpu.sync_copy(big_idx.at[pl.ds(prow_n, 6 * EB)], bidx_next)
            _softmax_batch(srcrowsA, dstrowsA, edgrowsA, scat, semA)
            scA = pltpu.async_copy(scat, s_acc.at[d0], semS, add=True)
            _issue(bidx_next, pstart + 2 * EB, 0,
                   srcrowsA, dstrowsA, edgrowsA, semA)
            pltpu.make_async_copy(edg_tab.at[pl.ds(idx_off, EB)],
                                  scatB, semS2).wait()
            _softmax_batch(srcrowsB, dstrowsB, edgrowsB, scatB, semB)
            scA.wait()
            pltpu.async_copy(scatB, s_acc.at[d1], semS2, add=True)

    pltpu.make_async_copy(src_tab.at[bidxA.at[pl.ds(0, EB)]],
                          srcrowsA, semA).wait()
    pltpu.make_async_copy(dst_tab.at[bidxA.at[pl.ds(2 * EB, EB)]],
                          dstrowsA, semA).wait()
    pltpu.make_async_copy(edg_tab.at[pl.ds(idx_off + ebase, EB)],
                          edgrowsA, semA).wait()
    pltpu.make_async_copy(edg_tab.at[pl.ds(idx_off, EB)],
                          scatB, semS2).wait()

    wid = si * NC + ci
    nbase = wid * (N_EDGES // (NC * NS))
    ne_row0 = nbase // (2 * EB)

    def _ne_issue(bidx, batch, rows_s, rows_d, sem):
        pltpu.async_copy(xe_tab.at[bidx.at[pl.ds(batch * EB, EB)]],
                         rows_s, sem)
        pltpu.async_copy(xe_tab.at[bidx.at[pl.ds((4 + batch) * EB, EB)]],
                         rows_d, sem)

    def _ne_drain(rows_s, rows_d, sem):
        pltpu.make_async_copy(xe_tab.at[bidxA.at[pl.ds(0, EB)]],
                              rows_s, sem).wait()
        pltpu.make_async_copy(xe_tab.at[bidxA.at[pl.ds(0, EB)]],
                              rows_d, sem).wait()

    def _ne_proc(rows_s, rows_d, start):
        pltpu.sync_copy(ee_tab.at[pl.ds(start, EB)], eerows)

        @pl.loop(0, EB)
        def _nbody(b):
            eerows[b, :] = (rows_s[b, pl.ds(0, EF)] +
                            rows_d[b, pl.ds(0, EF)] + eerows[b, :])

        pltpu.sync_copy(eerows, out_e.at[pl.ds(start, EB)])

    pltpu.sync_copy(big_idx.at[pl.ds(ne_row0 * (6 * EB), 6 * EB)], bidxA)
    _ne_issue(bidxA, 0, srcrowsA, dstrowsA, semA)

    @pl.loop(0, N_EDGES // (NC * NS * EB * 2))
    def _nepair(g):
        start = nbase + g * (2 * EB)
        _ne_issue(bidxA, 1, srcrowsB, dstrowsB, semB)
        prow_n = (ne_row0 + g + 1) * (6 * EB)
        pltpu.sync_copy(big_idx.at[pl.ds(prow_n, 6 * EB)], bidxB)
        _ne_drain(srcrowsA, dstrowsA, semA)
        _ne_proc(srcrowsA, dstrowsA, start)
        _ne_issue(bidxB, 0, srcrowsA, dstrowsA, semA)
        _ne_drain(srcrowsB, dstrowsB, semB)
        _ne_proc(srcrowsB, dstrowsB, start + EB)
        for c in range(0, 6 * EB, 16):
            bidxA[pl.ds(c, 16)] = bidxB[pl.ds(c, 16)]

    _ne_drain(srcrowsA, dstrowsA, semA)

    plsc.subcore_barrier()

    eps = jnp.full((16,), 1e-16, jnp.float32)
    bts = [btvec[pl.ds(mbase + h * 16, 16)] for h in range(4)]
    for k in range(NODES_PER_TILE // EB):
        base = si * NODES_PER_TILE + k * EB
        pltpu.sync_copy(s_acc.at[pl.ds(base, EB)], srcrowsA)

        @pl.loop(0, EB)
        def _ebody(r):
            for h in range(4):
                co = h * 16
                s0 = srcrowsA[r, pl.ds(co, 16)]
                s1 = srcrowsA[r, pl.ds(co + H, 16)]
                scat[r, pl.ds(co, 16)] = s1 / (s0 + eps) + bts[h]
                scat[r, pl.ds(co + H, 16)] = s0

        pltpu.sync_copy(scat, out_x.at[pl.ds(ci * NPAD + base, EB)])


_sc_pass = functools.partial(
    pl.kernel,
    out_type=[
        jax.ShapeDtypeStruct((NC * NPAD, D), jnp.float32),
        jax.ShapeDtypeStruct((N_EDGES, EF), jnp.float32),
    ],
    mesh=plsc.VectorSubcoreMesh(
        core_axis_name="c", subcore_axis_name="s", num_cores=NC,
        num_subcores=NS),
    scratch_types=[
        pltpu.VMEM_SHARED((NPAD, D), jnp.float32),
        pltpu.VMEM((6 * EB,), jnp.int32),
        pltpu.VMEM((6 * EB,), jnp.int32),
        pltpu.VMEM((EB,), jnp.int32),
        pltpu.VMEM((EB,), jnp.int32),
        pltpu.VMEM((EB,), jnp.int32),
        pltpu.VMEM((EB,), jnp.int32),
        pltpu.VMEM((EB, D), jnp.float32),
        pltpu.VMEM((EB, D), jnp.float32),
        pltpu.VMEM((EB, D), jnp.float32),
        pltpu.VMEM((EB, D), jnp.float32),
        pltpu.VMEM((EB, D), jnp.float32),
        pltpu.VMEM((EB, D), jnp.float32),
        pltpu.VMEM((EB, D), jnp.float32),
        pltpu.VMEM((EB, D), jnp.float32),
        pltpu.VMEM((EB, EF), jnp.float32),
        pltpu.VMEM((D,), jnp.float32),
        pltpu.VMEM((D,), jnp.float32),
        pltpu.VMEM((16,), jnp.float32),
        pltpu.SemaphoreType.DMA,
        pltpu.SemaphoreType.DMA,
        pltpu.SemaphoreType.DMA,
        pltpu.SemaphoreType.DMA,
    ],
)(_sc_body)


def kernel(x, edge_index, edge_attr, W_a, W_T, b_T, W_e, W_ee, prelu_w):
    x = x.astype(jnp.float32)
    e = edge_attr.astype(jnp.float32)
    src = edge_index[0].astype(jnp.int32)
    dst = edge_index[1].astype(jnp.int32)

    A2, Ae, A1 = W_a[:V_IN], W_a[V_IN:V_IN + EF], W_a[V_IN + EF:]
    T2, Te, T1 = W_T[:V_IN], W_T[V_IN:V_IN + EF], W_T[V_IN + EF:]

    def halves(a_part, t_part):
        return jnp.stack([
            jnp.concatenate([a_part[:, :H], t_part[:, :H]], axis=1),
            jnp.concatenate([a_part[:, H:], t_part[:, H:]], axis=1),
        ])

    ws = halves(A1, T1)
    wd = halves(A2, T2)
    wa = halves(Ae, Te)

    wep = jnp.zeros((V_IN, D), jnp.float32).at[:, :EF].set(W_e)
    src_pair, dst_pair, xe = _node_tables(x, ws, wd, wep)
    edg_pair, ee, amx, amn = _edge_tables(e, wa, W_ee)

    smax = jnp.concatenate([src_pair[0, :, :H].max(0), src_pair[1, :, :H].max(0)])
    smin = jnp.concatenate([src_pair[0, :, :H].min(0), src_pair[1, :, :H].min(0)])
    dmax = jnp.concatenate([dst_pair[0, :, :H].max(0), dst_pair[1, :, :H].max(0)])
    dmin = jnp.concatenate([dst_pair[0, :, :H].min(0), dst_pair[1, :, :H].min(0)])
    emax = amx.max(axis=(0, 1))
    emin = amn.min(axis=(0, 1))
    hi = smax + dmax + emax
    lo = smin + dmin + emin
    mvec = jnp.maximum(hi, jnp.maximum(prelu_w * hi, prelu_w * lo))
    mvec = mvec.astype(jnp.float32)

    src_tab = src_pair.reshape(NC * N_NODES, D)
    dst_tab = dst_pair.reshape(NC * N_NODES, D)
    edg_tab = edg_pair.reshape(NC * N_EDGES, D)
    pwv = jnp.full((16,), prelu_w, jnp.float32)
    srcp = src.reshape(-1, 2 * EB)
    dstp = dst.reshape(-1, 2 * EB)
    big = jnp.concatenate(
        [jnp.concatenate([srcp + h2 * N_NODES, dstp + h2 * N_NODES, dstp],
                         axis=1) for h2 in range(NC)]).reshape(-1)
    big = jnp.concatenate([big, jnp.zeros((6 * EB,), jnp.int32)])

    out_x, out_e = _sc_pass(src_tab, dst_tab, edg_tab, xe, ee, big,
                            mvec, b_T.astype(jnp.float32), pwv)

    new_x = jnp.concatenate([out_x[:N_NODES, :H], out_x[NPAD:NPAD + N_NODES, :H]],
                            axis=1)
    return (new_x, out_e)
